# Initial kernel scaffold; baseline (speedup 1.0000x reference)
import functools

import jax
import jax.numpy as jnp
from jax import lax
from jax.experimental import pallas as pl
from jax.experimental.pallas import tpu as pltpu
from jax.experimental.pallas import tpu_sc as plsc

NC = 2
NS = 16
LN = 16
NW = NC * NS

F32 = jnp.float32
I32 = jnp.int32


def _round_up(v, m):
    return (v + m - 1) // m * m


def _mesh():
    return plsc.VectorSubcoreMesh(core_axis_name="c", subcore_axis_name="s")


def _tc_matmul_pad(x, W, NP):
    N, D = x.shape

    def body(x_ref, w_ref, o_ref):
        o_ref[:N] = jnp.dot(x_ref[...], w_ref[...],
                            preferred_element_type=F32)
        o_ref[N:] = jnp.zeros((NP - N, D), F32)

    return pl.pallas_call(
        body, out_shape=jax.ShapeDtypeStruct((NP, D), F32))(x, W)


def _tc_mid(y1, asum, cnt, b1, g1, be1, W2, N):
    NP, D = y1.shape

    def body(y_ref, a_ref, c_ref, b1_ref, g1_ref, be1_ref, w2_ref,
             y2_ref, deg_ref):
        deg = jnp.clip(c_ref[0] + c_ref[1], 1.0, None)
        agg = (a_ref[0] + a_ref[1]) / deg[:, None]
        h = y_ref[...] + agg + b1_ref[...][None, :]
        hN = h[:N]
        mu = jnp.mean(hN, axis=0)
        var = jnp.mean((hN - mu[None, :]) ** 2, axis=0)
        hn = (h - mu[None, :]) * lax.rsqrt(var + 1e-5)[None, :]
        hn = hn * g1_ref[...][None, :] + be1_ref[...][None, :]
        hr = jnp.maximum(hn, 0.0)
        y2_ref[...] = jnp.dot(hr, w2_ref[...], preferred_element_type=F32)
        deg_ref[...] = deg

    return pl.pallas_call(
        body,
        out_shape=[jax.ShapeDtypeStruct((NP, D), F32),
                   jax.ShapeDtypeStruct((NP,), F32)],
    )(y1, asum, cnt, b1, g1, be1, W2)


def _tc_pq(y2, asum, degc, b2, w3a, w3b):
    NP, D = y2.shape

    def body(y_ref, a_ref, d_ref, b2_ref, wa_ref, wb_ref, p_ref, q_ref):
        agg = (a_ref[0] + a_ref[1]) / d_ref[...][:, None]
        h2 = jnp.maximum(y_ref[...] + agg + b2_ref[...][None, :], 0.0)
        p_ref[...] = jnp.sum(h2 * wa_ref[...][None, :], axis=1)
        q_ref[...] = jnp.sum(h2 * wb_ref[...][None, :], axis=1)

    return pl.pallas_call(
        body,
        out_shape=[jax.ShapeDtypeStruct((NP,), F32),
                   jax.ShapeDtypeStruct((NP,), F32)],
    )(y2, asum, degc, b2, w3a, w3b)


def _sc_degree(dstp, NP):
    EP = dstp.shape[0]
    ET = EP // NW
    NCH = ET // 128
    NPr = NP // LN

    @functools.partial(
        pl.kernel,
        out_type=jax.ShapeDtypeStruct((NC, NPr, LN), F32),
        mesh=_mesh(),
        scratch_types=[
            pltpu.VMEM((ET,), I32),
            pltpu.VMEM((128, LN), F32),
            pltpu.VMEM((1, 128), I32),
            pltpu.VMEM((NPr // NS, LN), F32),
            pltpu.VMEM_SHARED((NPr, LN), F32),
        ],
    )
    def k(dst_hbm, out_hbm, dst1, upd, rowb, zbuf, acc_sh):
        c = lax.axis_index("c")
        sid = lax.axis_index("s")
        wid = c * NS + sid
        stripe = NPr // NS

        iota = lax.iota(I32, LN)
        zero16 = jnp.zeros((LN,), F32)
        ones16 = jnp.ones((LN,), F32)

        for r in range(stripe):
            zbuf[r] = zero16
        pltpu.sync_copy(zbuf, acc_sh.at[pl.ds(sid * stripe, stripe)])
        pltpu.sync_copy(dst_hbm.at[pl.ds(wid * ET, ET)], dst1)
        plsc.subcore_barrier()

        @pl.loop(0, NCH)
        def _(ch):
            for g in range(8):
                base = ch * 128 + g * 16
                dv = dst1[pl.ds(base, LN)]
                row = lax.shift_right_logical(dv, 4)
                lane = lax.bitwise_and(dv, 15)
                for r in range(16):
                    upd[g * 16 + r] = zero16
                plsc.store_scatter(upd, [g * 16 + iota, lane], ones16)
                rowb[0, pl.ds(g * 16, LN)] = row
            pltpu.sync_copy(upd, acc_sh.at[rowb.at[0]], add=True)

        plsc.subcore_barrier()
        pltpu.sync_copy(acc_sh.at[pl.ds(sid * stripe, stripe)], zbuf)
        pltpu.sync_copy(zbuf, out_hbm.at[c, pl.ds(sid * stripe, stripe)])

    return k(dstp)


def _sc_rowscatter(y, src2h, dst2h, NP):
    D = y.shape[1]
    EPc = src2h.shape[0]
    NCH = EPc // NW
    stripe = NP // NS

    @functools.partial(
        pl.kernel,
        out_type=jax.ShapeDtypeStruct((NC, NP, D), F32),
        mesh=_mesh(),
        scratch_types=[
            pltpu.VMEM((NCH, 128), I32),
            pltpu.VMEM((NCH, 128), I32),
            pltpu.VMEM((128, D), F32),
            pltpu.VMEM_SHARED((NP, D), F32),
        ],
    )
    def k(y_hbm, src_hbm, dst_hbm, out_hbm, src2, dst2, rows, acc_sh):
        c = lax.axis_index("c")
        sid = lax.axis_index("s")
        wid = c * NS + sid

        zero16 = jnp.zeros((LN,), F32)

        @pl.loop(0, 128)
        def _(r):
            for kk in range(D // LN):
                rows[r, pl.ds(kk * LN, LN)] = zero16

        for t in range(stripe // 128):
            pltpu.sync_copy(rows,
                            acc_sh.at[pl.ds(sid * stripe + t * 128, 128)])

        pltpu.sync_copy(src_hbm.at[pl.ds(wid * NCH, NCH)], src2)
        pltpu.sync_copy(dst_hbm.at[pl.ds(wid * NCH, NCH)], dst2)
        plsc.subcore_barrier()

        @pl.loop(0, NCH)
        def _(ch):
            pltpu.sync_copy(y_hbm.at[src2.at[ch]], rows)
            pltpu.sync_copy(rows, acc_sh.at[dst2.at[ch]], add=True)

        plsc.subcore_barrier()
        for t in range(stripe // 128):
            pltpu.sync_copy(acc_sh.at[pl.ds(sid * stripe + t * 128, 128)],
                            rows)
            pltpu.sync_copy(rows,
                            out_hbm.at[c, pl.ds(sid * stripe + t * 128, 128)])

    return k(y, src2h, dst2h)


def _sc_edge_logits(p, q, srcp, dstp, NP):
    EP = srcp.shape[0]
    ET = EP // NW
    NCH = ET // 128
    NPr = NP // LN

    @functools.partial(
        pl.kernel,
        out_type=[jax.ShapeDtypeStruct((EP,), F32),
                  jax.ShapeDtypeStruct((NC, NPr, LN), F32)],
        mesh=_mesh(),
        scratch_types=[
            pltpu.VMEM((NP,), F32),
            pltpu.VMEM((NP,), F32),
            pltpu.VMEM((ET,), I32),
            pltpu.VMEM((ET,), I32),
            pltpu.VMEM((ET,), F32),
            pltpu.VMEM((128, LN), F32),
            pltpu.VMEM((1, 128), I32),
            pltpu.VMEM((NPr // NS, LN), F32),
            pltpu.VMEM_SHARED((NPr, LN), F32),
        ],
    )
    def k(p_hbm, q_hbm, src_hbm, dst_hbm, L_hbm, s_hbm,
          pt, qt, src1, dst1, Lt, upd, rowb, zbuf, acc_sh):
        c = lax.axis_index("c")
        sid = lax.axis_index("s")
        wid = c * NS + sid
        stripe = NPr // NS

        iota = lax.iota(I32, LN)
        zero16 = jnp.zeros((LN,), F32)

        for r in range(stripe):
            zbuf[r] = zero16
        pltpu.sync_copy(zbuf, acc_sh.at[pl.ds(sid * stripe, stripe)])
        pltpu.sync_copy(p_hbm, pt)
        pltpu.sync_copy(q_hbm, qt)
        pltpu.sync_copy(src_hbm.at[pl.ds(wid * ET, ET)], src1)
        pltpu.sync_copy(dst_hbm.at[pl.ds(wid * ET, ET)], dst1)
        plsc.subcore_barrier()

        @pl.loop(0, NCH)
        def _(ch):
            for g in range(8):
                base = ch * 128 + g * 16
                sv = src1[pl.ds(base, LN)]
                dv = dst1[pl.ds(base, LN)]
                pv = plsc.load_gather(pt, [sv])
                qv = plsc.load_gather(qt, [dv])
                Lv = pv + qv
                Lt[pl.ds(base, LN)] = Lv
                row = lax.shift_right_logical(dv, 4)
                lane = lax.bitwise_and(dv, 15)
                for r in range(16):
                    upd[g * 16 + r] = zero16
                plsc.store_scatter(upd, [g * 16 + iota, lane], Lv)
                rowb[0, pl.ds(g * 16, LN)] = row
            pltpu.sync_copy(upd, acc_sh.at[rowb.at[0]], add=True)

        pltpu.sync_copy(Lt, L_hbm.at[pl.ds(wid * ET, ET)])
        plsc.subcore_barrier()
        pltpu.sync_copy(acc_sh.at[pl.ds(sid * stripe, stripe)], zbuf)
        pltpu.sync_copy(zbuf, s_hbm.at[c, pl.ds(sid * stripe, stripe)])

    return k(p, q, srcp, dstp)


def _sc_final(L, srcp, dstp, s, degc, b3b):
    EP = srcp.shape[0]
    NP = degc.shape[0]
    ET = EP // NW

    @functools.partial(
        pl.kernel,
        out_type=jax.ShapeDtypeStruct((EP,), F32),
        mesh=_mesh(),
        scratch_types=[
            pltpu.VMEM((NP,), F32),
            pltpu.VMEM((NP,), F32),
            pltpu.VMEM((NP,), F32),
            pltpu.VMEM((ET,), I32),
            pltpu.VMEM((ET,), I32),
            pltpu.VMEM((ET,), F32),
            pltpu.VMEM((ET,), F32),
            pltpu.VMEM((LN,), F32),
        ],
    )
    def k(L_hbm, src_hbm, dst_hbm, s_hbm, deg_hbm, b3_hbm, out_hbm,
          st, tmp, degt, src1, dst1, Lt, outt, b3t):
        c = lax.axis_index("c")
        sid = lax.axis_index("s")
        wid = c * NS + sid

        pltpu.sync_copy(s_hbm.at[0], st)
        pltpu.sync_copy(s_hbm.at[1], tmp)
        pltpu.sync_copy(deg_hbm, degt)
        pltpu.sync_copy(src_hbm.at[pl.ds(wid * ET, ET)], src1)
        pltpu.sync_copy(dst_hbm.at[pl.ds(wid * ET, ET)], dst1)
        pltpu.sync_copy(L_hbm.at[pl.ds(wid * ET, ET)], Lt)
        pltpu.sync_copy(b3_hbm, b3t)

        @pl.loop(0, NP, step=LN)
        def _(j):
            st[pl.ds(j, LN)] = st[pl.ds(j, LN)] + tmp[pl.ds(j, LN)]

        b3v = b3t[...]
        one = jnp.ones((LN,), F32)
        zero = jnp.zeros((LN,), F32)

        @pl.loop(0, ET, step=LN)
        def _(j):
            sv = src1[pl.ds(j, LN)]
            dv = dst1[pl.ds(j, LN)]
            Lv = Lt[pl.ds(j, LN)]
            selfm = sv == dv
            s_at = plsc.load_gather(st, [sv])
            d_at = plsc.load_gather(degt, [sv])
            degv = jnp.maximum(d_at - jnp.where(selfm, one, zero), 1.0)
            num = s_at - jnp.where(selfm, Lv, zero)
            z = Lv + num / degv + b3v
            outt[pl.ds(j, LN)] = 1.0 / (1.0 + jnp.exp(-z))

        pltpu.sync_copy(outt, out_hbm.at[pl.ds(wid * ET, ET)])

    return k(L, srcp, dstp, s, degc, b3b)


def kernel(x, edge_index, W1, b1, gamma1, beta1, W2, b2, W3, b3):
    N, D = x.shape
    E = edge_index.shape[1]
    NP = _round_up(N, 256)
    EP = _round_up(E, 128 * NW)

    src = edge_index[0]
    dst = edge_index[1]
    npad = EP - E
    padv = N + (jnp.arange(npad, dtype=I32) % (NP - N))
    srcp = jnp.concatenate([src, padv])
    dstp = jnp.concatenate([dst, padv])
    src2h = srcp.reshape(EP // 128, 128)
    dst2h = dstp.reshape(EP // 128, 128)

    w3a = W3[:D, 0]
    w3b = W3[D:, 0]
    b3b = jnp.broadcast_to(b3, (LN,)).astype(F32)

    cnt_parts = _sc_degree(dstp, NP)
    y1 = _tc_matmul_pad(x, W1, NP)
    asum1 = _sc_rowscatter(y1, src2h, dst2h, NP)
    cnt = cnt_parts.reshape(NC, NP)
    y2, degc = _tc_mid(y1, asum1, cnt, b1, gamma1, beta1, W2, N)
    asum2 = _sc_rowscatter(y2, src2h, dst2h, NP)
    p, q = _tc_pq(y2, asum2, degc, b2, w3a, w3b)
    L, s_parts = _sc_edge_logits(p, q, srcp, dstp, NP)
    s = s_parts.reshape(NC, NP)
    outp = _sc_final(L, srcp, dstp, s, degc, b3b)
    return outp[:E][:, None]


# speedup vs baseline: 9.8364x; 9.8364x over previous
import dataclasses
import functools

import jax
import jax.numpy as jnp
from jax import lax
from jax.experimental import pallas as pl
from jax.experimental.pallas import tpu as pltpu
from jax.experimental.pallas import tpu_sc as plsc

NC = 2
NS = 16
LN = 16
NW = NC * NS

F32 = jnp.float32
I32 = jnp.int32


def _round_up(v, m):
    return (v + m - 1) // m * m


def _mesh():
    return plsc.VectorSubcoreMesh(core_axis_name="c", subcore_axis_name="s")


def _sc_params():
    cp = pltpu.CompilerParams()
    if "needs_layout_passes" in pltpu.CompilerParams.__dataclass_fields__:
        cp = dataclasses.replace(cp, needs_layout_passes=False)
    return cp


def _tc_matmul_pad(x, W, NP):
    N, D = x.shape

    def body(x_ref, w_ref, o_ref):
        o_ref[:N] = jnp.dot(x_ref[...], w_ref[...],
                            preferred_element_type=F32)
        o_ref[N:] = jnp.zeros((NP - N, D), F32)

    return pl.pallas_call(
        body, out_shape=jax.ShapeDtypeStruct((NP, D), F32))(x, W)


def _tc_mid(y1, asum, cnt, b1, g1, be1, W2, N):
    NP, D = y1.shape

    def body(y_ref, a_ref, c_ref, b1_ref, g1_ref, be1_ref, w2_ref,
             y2_ref, deg_ref):
        deg = jnp.clip(c_ref[0] + c_ref[1], 1.0, None)
        agg = (a_ref[0] + a_ref[1]) / deg[:, None]
        h = y_ref[...] + agg + b1_ref[...][None, :]
        hN = h[:N]
        mu = jnp.mean(hN, axis=0)
        var = jnp.mean((hN - mu[None, :]) ** 2, axis=0)
        hn = (h - mu[None, :]) * lax.rsqrt(var + 1e-5)[None, :]
        hn = hn * g1_ref[...][None, :] + be1_ref[...][None, :]
        hr = jnp.maximum(hn, 0.0)
        y2_ref[...] = jnp.dot(hr, w2_ref[...], preferred_element_type=F32)
        deg_ref[...] = deg

    return pl.pallas_call(
        body,
        out_shape=[jax.ShapeDtypeStruct((NP, D), F32),
                   jax.ShapeDtypeStruct((NP,), F32)],
    )(y1, asum, cnt, b1, g1, be1, W2)


def _tc_pq(y2, asum, degc, b2, w3a, w3b):
    NP, D = y2.shape

    def body(y_ref, a_ref, d_ref, b2_ref, wa_ref, wb_ref, p_ref, q_ref):
        agg = (a_ref[0] + a_ref[1]) / d_ref[...][:, None]
        h2 = jnp.maximum(y_ref[...] + agg + b2_ref[...][None, :], 0.0)
        p_ref[...] = jnp.sum(h2 * wa_ref[...][None, :], axis=1)
        q_ref[...] = jnp.sum(h2 * wb_ref[...][None, :], axis=1)

    return pl.pallas_call(
        body,
        out_shape=[jax.ShapeDtypeStruct((NP,), F32),
                   jax.ShapeDtypeStruct((NP,), F32)],
    )(y2, asum, degc, b2, w3a, w3b)


def _sc_degree(dstp, NP):
    EP = dstp.shape[0]
    NCH = EP // NW // 128
    stripe = NP // NS

    @functools.partial(
        pl.kernel,
        out_type=jax.ShapeDtypeStruct((NC * NP,), F32),
        mesh=_mesh(),
        compiler_params=_sc_params(),
        scratch_types=[
            pltpu.VMEM((NCH, 128), I32),
            pltpu.VMEM((128,), F32),
            pltpu.VMEM((stripe,), F32),
            pltpu.VMEM_SHARED((NP,), F32),
        ],
    )
    def k(dst_hbm, out_hbm, dst2, ones, zbuf, acc_sh):
        c = lax.axis_index("c")
        sid = lax.axis_index("s")
        wid = c * NS + sid

        zero16 = jnp.zeros((LN,), F32)
        ones16 = jnp.ones((LN,), F32)

        @pl.loop(0, stripe, step=LN)
        def _(j):
            zbuf[pl.ds(j, LN)] = zero16

        @pl.loop(0, 128, step=LN)
        def _(j):
            ones[pl.ds(j, LN)] = ones16

        pltpu.sync_copy(zbuf, acc_sh.at[pl.ds(sid * stripe, stripe)])

        @pl.loop(0, NCH)
        def _(t):
            pltpu.sync_copy(dst_hbm.at[pl.ds(wid * NCH * 128 + t * 128, 128)],
                            dst2.at[t])

        plsc.subcore_barrier()

        @pl.loop(0, NCH)
        def _(ch):
            pltpu.sync_copy(ones, acc_sh.at[dst2.at[ch]], add=True)

        plsc.subcore_barrier()
        pltpu.sync_copy(acc_sh.at[pl.ds(sid * stripe, stripe)], zbuf)
        pltpu.sync_copy(zbuf,
                        out_hbm.at[pl.ds(c * NP + sid * stripe, stripe)])

    return k(dstp)


def _sc_rowscatter(y, srcp, dstp, NP):
    D = y.shape[1]
    EP = srcp.shape[0]
    NCH = EP // NW // 128
    stripe = NP // NS

    @functools.partial(
        pl.kernel,
        out_type=jax.ShapeDtypeStruct((NC, NP, D), F32),
        mesh=_mesh(),
        compiler_params=_sc_params(),
        scratch_types=[
            pltpu.VMEM((NCH, 128), I32),
            pltpu.VMEM((NCH, 128), I32),
            pltpu.VMEM((128, D), F32),
            pltpu.VMEM_SHARED((NP, D), F32),
        ],
    )
    def k(y_hbm, src_hbm, dst_hbm, out_hbm, src2, dst2, rows, acc_sh):
        c = lax.axis_index("c")
        sid = lax.axis_index("s")
        wid = c * NS + sid

        zero16 = jnp.zeros((LN,), F32)

        @pl.loop(0, 128)
        def _(r):
            for kk in range(D // LN):
                rows[r, pl.ds(kk * LN, LN)] = zero16

        for t in range(stripe // 128):
            pltpu.sync_copy(rows,
                            acc_sh.at[pl.ds(sid * stripe + t * 128, 128)])

        @pl.loop(0, NCH)
        def _(t):
            pltpu.sync_copy(src_hbm.at[pl.ds(wid * NCH * 128 + t * 128, 128)],
                            src2.at[t])
            pltpu.sync_copy(dst_hbm.at[pl.ds(wid * NCH * 128 + t * 128, 128)],
                            dst2.at[t])

        plsc.subcore_barrier()

        @pl.loop(0, NCH)
        def _(ch):
            pltpu.sync_copy(y_hbm.at[src2.at[ch]], rows)
            pltpu.sync_copy(rows, acc_sh.at[dst2.at[ch]], add=True)

        plsc.subcore_barrier()
        for t in range(stripe // 128):
            pltpu.sync_copy(acc_sh.at[pl.ds(sid * stripe + t * 128, 128)],
                            rows)
            pltpu.sync_copy(rows,
                            out_hbm.at[c, pl.ds(sid * stripe + t * 128, 128)])

    return k(y, srcp, dstp)


def _sc_edge_logits(p, q, srcp, dstp, NP):
    EP = srcp.shape[0]
    ET = EP // NW
    NCH = ET // 128
    stripe = NP // NS

    @functools.partial(
        pl.kernel,
        out_type=[jax.ShapeDtypeStruct((EP,), F32),
                  jax.ShapeDtypeStruct((NC * NP,), F32)],
        mesh=_mesh(),
        compiler_params=_sc_params(),
        scratch_types=[
            pltpu.VMEM((NP,), F32),
            pltpu.VMEM((NP,), F32),
            pltpu.VMEM((ET,), I32),
            pltpu.VMEM((ET,), I32),
            pltpu.VMEM((NCH, 128), I32),
            pltpu.VMEM((ET,), F32),
            pltpu.VMEM((stripe,), F32),
            pltpu.VMEM_SHARED((NP,), F32),
        ],
    )
    def k(p_hbm, q_hbm, src_hbm, dst_hbm, L_hbm, s_hbm,
          pt, qt, src1, dst1, dst2, Lt, zbuf, acc_sh):
        c = lax.axis_index("c")
        sid = lax.axis_index("s")
        wid = c * NS + sid

        zero16 = jnp.zeros((LN,), F32)

        @pl.loop(0, stripe, step=LN)
        def _(j):
            zbuf[pl.ds(j, LN)] = zero16

        pltpu.sync_copy(zbuf, acc_sh.at[pl.ds(sid * stripe, stripe)])
        pltpu.sync_copy(p_hbm, pt)
        pltpu.sync_copy(q_hbm, qt)
        pltpu.sync_copy(src_hbm.at[pl.ds(wid * ET, ET)], src1)
        pltpu.sync_copy(dst_hbm.at[pl.ds(wid * ET, ET)], dst1)

        @pl.loop(0, NCH)
        def _(t):
            pltpu.sync_copy(dst_hbm.at[pl.ds(wid * ET + t * 128, 128)],
                            dst2.at[t])

        plsc.subcore_barrier()

        @pl.loop(0, NCH)
        def _(ch):
            for g in range(8):
                base = ch * 128 + g * 16
                sv = src1[pl.ds(base, LN)]
                dv = dst1[pl.ds(base, LN)]
                pv = plsc.load_gather(pt, [sv])
                qv = plsc.load_gather(qt, [dv])
                Lt[pl.ds(base, LN)] = pv + qv
            pltpu.sync_copy(Lt.at[pl.ds(ch * 128, 128)],
                            acc_sh.at[dst2.at[ch]], add=True)

        pltpu.sync_copy(Lt, L_hbm.at[pl.ds(wid * ET, ET)])
        plsc.subcore_barrier()
        pltpu.sync_copy(acc_sh.at[pl.ds(sid * stripe, stripe)], zbuf)
        pltpu.sync_copy(zbuf,
                        s_hbm.at[pl.ds(c * NP + sid * stripe, stripe)])

    return k(p, q, srcp, dstp)


def _sc_final(L, srcp, dstp, s2, degc, b3b):
    EP = srcp.shape[0]
    NP = degc.shape[0]
    ET = EP // NW

    @functools.partial(
        pl.kernel,
        out_type=jax.ShapeDtypeStruct((EP,), F32),
        mesh=_mesh(),
        compiler_params=_sc_params(),
        scratch_types=[
            pltpu.VMEM((NP,), F32),
            pltpu.VMEM((NP,), F32),
            pltpu.VMEM((NP,), F32),
            pltpu.VMEM((ET,), I32),
            pltpu.VMEM((ET,), I32),
            pltpu.VMEM((ET,), F32),
            pltpu.VMEM((ET,), F32),
            pltpu.VMEM((LN,), F32),
        ],
    )
    def k(L_hbm, src_hbm, dst_hbm, s_hbm, deg_hbm, b3_hbm, out_hbm,
          st, tmp, degt, src1, dst1, Lt, outt, b3t):
        c = lax.axis_index("c")
        sid = lax.axis_index("s")
        wid = c * NS + sid

        pltpu.sync_copy(s_hbm.at[pl.ds(0, NP)], st)
        pltpu.sync_copy(s_hbm.at[pl.ds(NP, NP)], tmp)
        pltpu.sync_copy(deg_hbm, degt)
        pltpu.sync_copy(src_hbm.at[pl.ds(wid * ET, ET)], src1)
        pltpu.sync_copy(dst_hbm.at[pl.ds(wid * ET, ET)], dst1)
        pltpu.sync_copy(L_hbm.at[pl.ds(wid * ET, ET)], Lt)
        pltpu.sync_copy(b3_hbm, b3t)

        @pl.loop(0, NP, step=LN)
        def _(j):
            st[pl.ds(j, LN)] = st[pl.ds(j, LN)] + tmp[pl.ds(j, LN)]

        b3v = b3t[...]
        one = jnp.ones((LN,), F32)
        zero = jnp.zeros((LN,), F32)

        @pl.loop(0, ET, step=LN)
        def _(j):
            sv = src1[pl.ds(j, LN)]
            dv = dst1[pl.ds(j, LN)]
            Lv = Lt[pl.ds(j, LN)]
            selfm = sv == dv
            s_at = plsc.load_gather(st, [sv])
            d_at = plsc.load_gather(degt, [sv])
            degv = jnp.maximum(d_at - jnp.where(selfm, one, zero), 1.0)
            num = s_at - jnp.where(selfm, Lv, zero)
            z = Lv + num / degv + b3v
            outt[pl.ds(j, LN)] = 1.0 / (1.0 + jnp.exp(-z))

        pltpu.sync_copy(outt, out_hbm.at[pl.ds(wid * ET, ET)])

    return k(L, srcp, dstp, s2, degc, b3b)


def kernel(x, edge_index, W1, b1, gamma1, beta1, W2, b2, W3, b3):
    N, D = x.shape
    E = edge_index.shape[1]
    NP = _round_up(N, 256)
    EP = _round_up(E, 128 * NW)

    src = edge_index[0]
    dst = edge_index[1]
    npad = EP - E
    padv = N + (jnp.arange(npad, dtype=I32) % (NP - N))
    srcp = jnp.concatenate([src, padv])
    dstp = jnp.concatenate([dst, padv])

    w3a = W3[:D, 0]
    w3b = W3[D:, 0]
    b3b = jnp.broadcast_to(b3, (LN,)).astype(F32)

    cnt2 = _sc_degree(dstp, NP)
    y1 = _tc_matmul_pad(x, W1, NP)
    asum1 = _sc_rowscatter(y1, srcp, dstp, NP)
    cnt = cnt2.reshape(NC, NP)
    y2, degc = _tc_mid(y1, asum1, cnt, b1, gamma1, beta1, W2, N)
    asum2 = _sc_rowscatter(y2, srcp, dstp, NP)
    p, q = _tc_pq(y2, asum2, degc, b2, w3a, w3b)
    L, s2 = _sc_edge_logits(p, q, srcp, dstp, NP)
    outp = _sc_final(L, srcp, dstp, s2, degc, b3b)
    return outp[:E][:, None]


# speedup vs baseline: 14.9618x; 1.5211x over previous
import dataclasses
import functools

import jax
import jax.numpy as jnp
from jax import lax
from jax.experimental import pallas as pl
from jax.experimental.pallas import tpu as pltpu
from jax.experimental.pallas import tpu_sc as plsc

NC = 2
NS = 16
LN = 16
NW = NC * NS

F32 = jnp.float32
I32 = jnp.int32


def _round_up(v, m):
    return (v + m - 1) // m * m


def _mesh():
    return plsc.VectorSubcoreMesh(core_axis_name="c", subcore_axis_name="s")


def _sc_params():
    cp = pltpu.CompilerParams()
    if "needs_layout_passes" in pltpu.CompilerParams.__dataclass_fields__:
        cp = dataclasses.replace(cp, needs_layout_passes=False)
    return cp


def _tc_matmul_pad(x, W, NP):
    N, D = x.shape

    def body(x_ref, w_ref, o_ref):
        o_ref[:N] = jnp.dot(x_ref[...], w_ref[...],
                            preferred_element_type=F32)
        o_ref[N:] = jnp.zeros((NP - N, D), F32)

    return pl.pallas_call(
        body, out_shape=jax.ShapeDtypeStruct((NP, D), F32))(x, W)


def _tc_mid(y1, asum, cnt, b1, g1, be1, W2, N):
    NP, D = y1.shape

    def body(y_ref, a_ref, c_ref, b1_ref, g1_ref, be1_ref, w2_ref,
             y2_ref, deg_ref):
        deg = jnp.clip(c_ref[0] + c_ref[1], 1.0, None)
        agg = (a_ref[0] + a_ref[1]) / deg[:, None]
        h = y_ref[...] + agg + b1_ref[...][None, :]
        hN = h[:N]
        mu = jnp.mean(hN, axis=0)
        var = jnp.mean((hN - mu[None, :]) ** 2, axis=0)
        hn = (h - mu[None, :]) * lax.rsqrt(var + 1e-5)[None, :]
        hn = hn * g1_ref[...][None, :] + be1_ref[...][None, :]
        hr = jnp.maximum(hn, 0.0)
        y2_ref[...] = jnp.dot(hr, w2_ref[...], preferred_element_type=F32)
        deg_ref[...] = deg

    return pl.pallas_call(
        body,
        out_shape=[jax.ShapeDtypeStruct((NP, D), F32),
                   jax.ShapeDtypeStruct((NP,), F32)],
    )(y1, asum, cnt, b1, g1, be1, W2)


def _tc_pq(y2, asum, degc, b2, w3a, w3b):
    NP, D = y2.shape

    def body(y_ref, a_ref, d_ref, b2_ref, wa_ref, wb_ref, p_ref, q_ref):
        agg = (a_ref[0] + a_ref[1]) / d_ref[...][:, None]
        h2 = jnp.maximum(y_ref[...] + agg + b2_ref[...][None, :], 0.0)
        p_ref[...] = jnp.sum(h2 * wa_ref[...][None, :], axis=1)
        q_ref[...] = jnp.sum(h2 * wb_ref[...][None, :], axis=1)

    return pl.pallas_call(
        body,
        out_shape=[jax.ShapeDtypeStruct((NP,), F32),
                   jax.ShapeDtypeStruct((NP,), F32)],
    )(y2, asum, degc, b2, w3a, w3b)


def _sc_rowscatter(y, srcp, dstp, NP, with_cnt=False):
    D = y.shape[1]
    EP = srcp.shape[0]
    NCH = EP // NW // 128
    stripe = NP // NS

    out_type = [jax.ShapeDtypeStruct((NC, NP, D), F32)]
    scratch = [
        pltpu.VMEM((NCH, 128), I32),
        pltpu.VMEM((NCH, 128), I32),
        pltpu.VMEM((128, D), F32),
        pltpu.VMEM((128, D), F32),
        pltpu.VMEM_SHARED((NP, D), F32),
        pltpu.SemaphoreType.DMA,
        pltpu.SemaphoreType.DMA,
        pltpu.SemaphoreType.DMA,
    ]
    if with_cnt:
        out_type.append(jax.ShapeDtypeStruct((NC * NP,), F32))
        scratch += [
            pltpu.VMEM((128,), F32),
            pltpu.VMEM((stripe,), F32),
            pltpu.VMEM_SHARED((NP,), F32),
        ]

    @functools.partial(
        pl.kernel,
        out_type=out_type,
        mesh=_mesh(),
        compiler_params=_sc_params(),
        scratch_types=scratch,
    )
    def k(y_hbm, src_hbm, dst_hbm, out_hbm, *rest):
        if with_cnt:
            (cnt_hbm, src2, dst2, rows0, rows1, acc_sh, g0, g1, st,
             ones, zbuf, cnt_sh) = rest
        else:
            src2, dst2, rows0, rows1, acc_sh, g0, g1, st = rest
        c = lax.axis_index("c")
        sid = lax.axis_index("s")
        wid = c * NS + sid
        bufs = (rows0, rows1)
        sems = (g0, g1)

        zero16 = jnp.zeros((LN,), F32)
        ones16 = jnp.ones((LN,), F32)

        @pl.loop(0, NCH)
        def _(t):
            base = wid * NCH * 128 + t * 128
            pltpu.async_copy(src_hbm.at[pl.ds(base, 128)], src2.at[t], st)
            pltpu.async_copy(dst_hbm.at[pl.ds(base, 128)], dst2.at[t], st)

        @pl.loop(0, 128)
        def _(r):
            for kk in range(D // LN):
                rows0[r, pl.ds(kk * LN, LN)] = zero16

        for t in range(stripe // 128):
            pltpu.sync_copy(rows0,
                            acc_sh.at[pl.ds(sid * stripe + t * 128, 128)])

        if with_cnt:
            @pl.loop(0, 128, step=LN)
            def _(j):
                ones[pl.ds(j, LN)] = ones16

            @pl.loop(0, stripe, step=LN)
            def _(j):
                zbuf[pl.ds(j, LN)] = zero16

            pltpu.sync_copy(zbuf, cnt_sh.at[pl.ds(sid * stripe, stripe)])

        @pl.loop(0, NCH)
        def _(t):
            pltpu.make_async_copy(src_hbm.at[pl.ds(0, 128)], src2.at[t],
                                  st).wait()
            pltpu.make_async_copy(dst_hbm.at[pl.ds(0, 128)], dst2.at[t],
                                  st).wait()

        plsc.subcore_barrier()

        pltpu.async_copy(y_hbm.at[src2.at[0]], rows0, g0)
        for i in range(NCH):
            buf = bufs[i % 2]
            pltpu.make_async_copy(y_hbm.at[src2.at[i]], buf,
                                  sems[i % 2]).wait()
            if i + 1 < NCH:
                pltpu.async_copy(y_hbm.at[src2.at[i + 1]],
                                 bufs[(i + 1) % 2], sems[(i + 1) % 2])
            pltpu.sync_copy(buf, acc_sh.at[dst2.at[i]], add=True)
            if with_cnt:
                pltpu.sync_copy(ones, cnt_sh.at[dst2.at[i]], add=True)

        plsc.subcore_barrier()
        for t in range(stripe // 128):
            pltpu.sync_copy(acc_sh.at[pl.ds(sid * stripe + t * 128, 128)],
                            rows0)
            pltpu.sync_copy(rows0,
                            out_hbm.at[c, pl.ds(sid * stripe + t * 128, 128)])
        if with_cnt:
            pltpu.sync_copy(cnt_sh.at[pl.ds(sid * stripe, stripe)], zbuf)
            pltpu.sync_copy(zbuf,
                            cnt_hbm.at[pl.ds(c * NP + sid * stripe, stripe)])

    return k(y, srcp, dstp)


def _sc_edge_logits(p, q, srcp, dstp, NP):
    EP = srcp.shape[0]
    ET = EP // NW
    NCH = ET // 128
    stripe = NP // NS

    @functools.partial(
        pl.kernel,
        out_type=[jax.ShapeDtypeStruct((EP,), F32),
                  jax.ShapeDtypeStruct((NC * NP,), F32)],
        mesh=_mesh(),
        compiler_params=_sc_params(),
        scratch_types=[
            pltpu.VMEM((NP,), F32),
            pltpu.VMEM((NP,), F32),
            pltpu.VMEM((ET,), I32),
            pltpu.VMEM((ET,), I32),
            pltpu.VMEM((NCH, 128), I32),
            pltpu.VMEM((ET,), F32),
            pltpu.VMEM((stripe,), F32),
            pltpu.VMEM_SHARED((NP,), F32),
        ],
    )
    def k(p_hbm, q_hbm, src_hbm, dst_hbm, L_hbm, s_hbm,
          pt, qt, src1, dst1, dst2, Lt, zbuf, acc_sh):
        c = lax.axis_index("c")
        sid = lax.axis_index("s")
        wid = c * NS + sid

        zero16 = jnp.zeros((LN,), F32)

        @pl.loop(0, stripe, step=LN)
        def _(j):
            zbuf[pl.ds(j, LN)] = zero16

        pltpu.sync_copy(zbuf, acc_sh.at[pl.ds(sid * stripe, stripe)])
        pltpu.sync_copy(p_hbm, pt)
        pltpu.sync_copy(q_hbm, qt)
        pltpu.sync_copy(src_hbm.at[pl.ds(wid * ET, ET)], src1)
        pltpu.sync_copy(dst_hbm.at[pl.ds(wid * ET, ET)], dst1)

        @pl.loop(0, NCH)
        def _(t):
            pltpu.sync_copy(dst_hbm.at[pl.ds(wid * ET + t * 128, 128)],
                            dst2.at[t])

        plsc.subcore_barrier()

        @pl.loop(0, NCH)
        def _(ch):
            for g in range(8):
                base = ch * 128 + g * 16
                sv = src1[pl.ds(base, LN)]
                dv = dst1[pl.ds(base, LN)]
                pv = plsc.load_gather(pt, [sv])
                qv = plsc.load_gather(qt, [dv])
                Lt[pl.ds(base, LN)] = pv + qv
            pltpu.sync_copy(Lt.at[pl.ds(ch * 128, 128)],
                            acc_sh.at[dst2.at[ch]], add=True)

        pltpu.sync_copy(Lt, L_hbm.at[pl.ds(wid * ET, ET)])
        plsc.subcore_barrier()
        pltpu.sync_copy(acc_sh.at[pl.ds(sid * stripe, stripe)], zbuf)
        pltpu.sync_copy(zbuf,
                        s_hbm.at[pl.ds(c * NP + sid * stripe, stripe)])

    return k(p, q, srcp, dstp)


def _sc_final(L, srcp, dstp, s2, degc, b3b):
    EP = srcp.shape[0]
    NP = degc.shape[0]
    ET = EP // NW

    @functools.partial(
        pl.kernel,
        out_type=jax.ShapeDtypeStruct((EP,), F32),
        mesh=_mesh(),
        compiler_params=_sc_params(),
        scratch_types=[
            pltpu.VMEM((NP,), F32),
            pltpu.VMEM((NP,), F32),
            pltpu.VMEM((NP,), F32),
            pltpu.VMEM((ET,), I32),
            pltpu.VMEM((ET,), I32),
            pltpu.VMEM((ET,), F32),
            pltpu.VMEM((ET,), F32),
            pltpu.VMEM((LN,), F32),
        ],
    )
    def k(L_hbm, src_hbm, dst_hbm, s_hbm, deg_hbm, b3_hbm, out_hbm,
          st, tmp, degt, src1, dst1, Lt, outt, b3t):
        c = lax.axis_index("c")
        sid = lax.axis_index("s")
        wid = c * NS + sid

        pltpu.sync_copy(s_hbm.at[pl.ds(0, NP)], st)
        pltpu.sync_copy(s_hbm.at[pl.ds(NP, NP)], tmp)
        pltpu.sync_copy(deg_hbm, degt)
        pltpu.sync_copy(src_hbm.at[pl.ds(wid * ET, ET)], src1)
        pltpu.sync_copy(dst_hbm.at[pl.ds(wid * ET, ET)], dst1)
        pltpu.sync_copy(L_hbm.at[pl.ds(wid * ET, ET)], Lt)
        pltpu.sync_copy(b3_hbm, b3t)

        @pl.loop(0, NP, step=LN)
        def _(j):
            st[pl.ds(j, LN)] = st[pl.ds(j, LN)] + tmp[pl.ds(j, LN)]

        b3v = b3t[...]
        one = jnp.ones((LN,), F32)
        zero = jnp.zeros((LN,), F32)

        @pl.loop(0, ET, step=LN)
        def _(j):
            sv = src1[pl.ds(j, LN)]
            dv = dst1[pl.ds(j, LN)]
            Lv = Lt[pl.ds(j, LN)]
            selfm = sv == dv
            s_at = plsc.load_gather(st, [sv])
            d_at = plsc.load_gather(degt, [sv])
            degv = jnp.maximum(d_at - jnp.where(selfm, one, zero), 1.0)
            num = s_at - jnp.where(selfm, Lv, zero)
            z = Lv + num / degv + b3v
            outt[pl.ds(j, LN)] = 1.0 / (1.0 + jnp.exp(-z))

        pltpu.sync_copy(outt, out_hbm.at[pl.ds(wid * ET, ET)])

    return k(L, srcp, dstp, s2, degc, b3b)


def kernel(x, edge_index, W1, b1, gamma1, beta1, W2, b2, W3, b3):
    N, D = x.shape
    E = edge_index.shape[1]
    NP = _round_up(N, 256)
    EP = _round_up(E, 128 * NW)

    src = edge_index[0]
    dst = edge_index[1]
    npad = EP - E
    padv = N + (jnp.arange(npad, dtype=I32) % (NP - N))
    srcp = jnp.concatenate([src, padv])
    dstp = jnp.concatenate([dst, padv])

    w3a = W3[:D, 0]
    w3b = W3[D:, 0]
    b3b = jnp.broadcast_to(b3, (LN,)).astype(F32)

    y1 = _tc_matmul_pad(x, W1, NP)
    asum1, cnt2 = _sc_rowscatter(y1, srcp, dstp, NP, with_cnt=True)
    cnt = cnt2.reshape(NC, NP)
    y2, degc = _tc_mid(y1, asum1, cnt, b1, gamma1, beta1, W2, N)
    asum2, = _sc_rowscatter(y2, srcp, dstp, NP)
    p, q = _tc_pq(y2, asum2, degc, b2, w3a, w3b)
    L, s2 = _sc_edge_logits(p, q, srcp, dstp, NP)
    outp = _sc_final(L, srcp, dstp, s2, degc, b3b)
    return outp[:E][:, None]


# speedup vs baseline: 16.6532x; 1.1130x over previous
import dataclasses
import functools

import jax
import jax.numpy as jnp
from jax import lax
from jax.experimental import pallas as pl
from jax.experimental.pallas import tpu as pltpu
from jax.experimental.pallas import tpu_sc as plsc

NC = 2
NS = 16
LN = 16
NW = NC * NS

F32 = jnp.float32
I32 = jnp.int32


def _round_up(v, m):
    return (v + m - 1) // m * m


def _mesh():
    return plsc.VectorSubcoreMesh(core_axis_name="c", subcore_axis_name="s")


def _sc_params():
    cp = pltpu.CompilerParams()
    if "needs_layout_passes" in pltpu.CompilerParams.__dataclass_fields__:
        cp = dataclasses.replace(cp, needs_layout_passes=False)
    return cp


def _tc_matmul_pad(x, W, NP):
    N, D = x.shape

    def body(x_ref, w_ref, o_ref):
        o_ref[:N] = jnp.dot(x_ref[...], w_ref[...],
                            preferred_element_type=F32)
        o_ref[N:] = jnp.zeros((NP - N, D), F32)

    return pl.pallas_call(
        body, out_shape=jax.ShapeDtypeStruct((NP, D), F32))(x, W)


def _tc_mid(y1, asum, cnt, b1, g1, be1, W2, N):
    NP, D = y1.shape

    def body(y_ref, a_ref, c_ref, b1_ref, g1_ref, be1_ref, w2_ref,
             y2_ref, deg_ref):
        deg = jnp.clip(c_ref[0] + c_ref[1], 1.0, None)
        agg = (a_ref[0] + a_ref[1]) / deg[:, None]
        h = y_ref[...] + agg + b1_ref[...][None, :]
        hN = h[:N]
        mu = jnp.mean(hN, axis=0)
        var = jnp.mean((hN - mu[None, :]) ** 2, axis=0)
        hn = (h - mu[None, :]) * lax.rsqrt(var + 1e-5)[None, :]
        hn = hn * g1_ref[...][None, :] + be1_ref[...][None, :]
        hr = jnp.maximum(hn, 0.0)
        y2_ref[...] = jnp.dot(hr, w2_ref[...], preferred_element_type=F32)
        deg_ref[...] = deg

    return pl.pallas_call(
        body,
        out_shape=[jax.ShapeDtypeStruct((NP, D), F32),
                   jax.ShapeDtypeStruct((NP,), F32)],
    )(y1, asum, cnt, b1, g1, be1, W2)


def _tc_pq(y2, asum, degc, b2, w3a, w3b):
    NP, D = y2.shape

    def body(y_ref, a_ref, d_ref, b2_ref, wa_ref, wb_ref, p_ref, q_ref):
        agg = (a_ref[0] + a_ref[1]) / d_ref[...][:, None]
        h2 = jnp.maximum(y_ref[...] + agg + b2_ref[...][None, :], 0.0)
        p_ref[...] = jnp.sum(h2 * wa_ref[...][None, :], axis=1)
        q_ref[...] = jnp.sum(h2 * wb_ref[...][None, :], axis=1)

    return pl.pallas_call(
        body,
        out_shape=[jax.ShapeDtypeStruct((NP,), F32),
                   jax.ShapeDtypeStruct((NP,), F32)],
    )(y2, asum, degc, b2, w3a, w3b)


def _sc_rowscatter(y, srcp, dstp, NP, with_cnt=False):
    D = y.shape[1]
    EP = srcp.shape[0]
    NCH = EP // NW // 128
    stripe = NP // NS

    NB = 2
    out_type = [jax.ShapeDtypeStruct((NC, NP, D), F32)]
    scratch = [
        pltpu.VMEM((NCH, 128), I32),
        pltpu.VMEM((NCH, 128), I32),
    ]
    scratch += [pltpu.VMEM((128, D), F32) for _ in range(NB)]
    scratch += [
        pltpu.VMEM_SHARED((NP, D), F32),
        pltpu.SemaphoreType.DMA,
    ]
    scratch += [pltpu.SemaphoreType.DMA for _ in range(NB)]
    scratch += [pltpu.SemaphoreType.DMA for _ in range(NB)]
    if with_cnt:
        out_type.append(jax.ShapeDtypeStruct((NC * NP,), F32))
        scratch += [
            pltpu.VMEM((128,), F32),
            pltpu.VMEM((stripe,), F32),
            pltpu.VMEM_SHARED((NP,), F32),
            pltpu.SemaphoreType.DMA,
        ]

    @functools.partial(
        pl.kernel,
        out_type=out_type,
        mesh=_mesh(),
        compiler_params=_sc_params(),
        scratch_types=scratch,
    )
    def k(y_hbm, src_hbm, dst_hbm, out_hbm, *rest):
        if with_cnt:
            cnt_hbm = rest[0]
            rest = rest[1:]
        src2, dst2 = rest[0], rest[1]
        bufs = rest[2:2 + NB]
        acc_sh = rest[2 + NB]
        st = rest[3 + NB]
        gsem = rest[4 + NB:4 + 2 * NB]
        ssem = rest[4 + 2 * NB:4 + 3 * NB]
        if with_cnt:
            ones, zbuf, cnt_sh, csem = rest[4 + 3 * NB:]
        c = lax.axis_index("c")
        sid = lax.axis_index("s")
        wid = c * NS + sid
        rows0 = bufs[0]

        zero16 = jnp.zeros((LN,), F32)
        ones16 = jnp.ones((LN,), F32)

        @pl.loop(0, NCH)
        def _(t):
            base = wid * NCH * 128 + t * 128
            pltpu.async_copy(src_hbm.at[pl.ds(base, 128)], src2.at[t], st)
            pltpu.async_copy(dst_hbm.at[pl.ds(base, 128)], dst2.at[t], st)

        @pl.loop(0, 128)
        def _(r):
            for kk in range(D // LN):
                rows0[r, pl.ds(kk * LN, LN)] = zero16

        for t in range(stripe // 128):
            pltpu.sync_copy(rows0,
                            acc_sh.at[pl.ds(sid * stripe + t * 128, 128)])

        if with_cnt:
            @pl.loop(0, 128, step=LN)
            def _(j):
                ones[pl.ds(j, LN)] = ones16

            @pl.loop(0, stripe, step=LN)
            def _(j):
                zbuf[pl.ds(j, LN)] = zero16

            pltpu.sync_copy(zbuf, cnt_sh.at[pl.ds(sid * stripe, stripe)])

        @pl.loop(0, NCH)
        def _(t):
            pltpu.make_async_copy(src_hbm.at[pl.ds(0, 128)], src2.at[t],
                                  st).wait()
            pltpu.make_async_copy(dst_hbm.at[pl.ds(0, 128)], dst2.at[t],
                                  st).wait()

        plsc.subcore_barrier()

        pltpu.async_copy(y_hbm.at[src2.at[0]], bufs[0], gsem[0])
        for i in range(NCH):
            b = i % NB
            pltpu.make_async_copy(y_hbm.at[src2.at[i]], bufs[b],
                                  gsem[b]).wait()
            pltpu.async_copy(bufs[b], acc_sh.at[dst2.at[i]], ssem[b],
                             add=True)
            if with_cnt:
                pltpu.async_copy(ones, cnt_sh.at[dst2.at[i]], csem,
                                 add=True)
            if i + 1 < NCH:
                nb = (i + 1) % NB
                if i - 1 >= 0:
                    pltpu.make_async_copy(bufs[nb], acc_sh.at[dst2.at[0]],
                                          ssem[nb]).wait()
                pltpu.async_copy(y_hbm.at[src2.at[i + 1]], bufs[nb],
                                 gsem[nb])
        for i in range(NCH - 2, NCH):
            pltpu.make_async_copy(bufs[i % NB], acc_sh.at[dst2.at[0]],
                                  ssem[i % NB]).wait()
        if with_cnt:
            for i in range(NCH):
                pltpu.make_async_copy(ones, cnt_sh.at[dst2.at[0]],
                                      csem).wait()

        plsc.subcore_barrier()
        for t in range(stripe // 128):
            pltpu.sync_copy(acc_sh.at[pl.ds(sid * stripe + t * 128, 128)],
                            rows0)
            pltpu.sync_copy(rows0,
                            out_hbm.at[c, pl.ds(sid * stripe + t * 128, 128)])
        if with_cnt:
            pltpu.sync_copy(cnt_sh.at[pl.ds(sid * stripe, stripe)], zbuf)
            pltpu.sync_copy(zbuf,
                            cnt_hbm.at[pl.ds(c * NP + sid * stripe, stripe)])

    return k(y, srcp, dstp)


def _sc_edge_logits(p, q, srcp, dstp, NP):
    EP = srcp.shape[0]
    ET = EP // NW
    NCH = ET // 128
    stripe = NP // NS

    @functools.partial(
        pl.kernel,
        out_type=[jax.ShapeDtypeStruct((EP,), F32),
                  jax.ShapeDtypeStruct((NC * NP,), F32)],
        mesh=_mesh(),
        compiler_params=_sc_params(),
        scratch_types=[
            pltpu.VMEM((NP,), F32),
            pltpu.VMEM((NP,), F32),
            pltpu.VMEM((ET,), I32),
            pltpu.VMEM((ET,), I32),
            pltpu.VMEM((NCH, 128), I32),
            pltpu.VMEM((ET,), F32),
            pltpu.VMEM((stripe,), F32),
            pltpu.VMEM_SHARED((NP,), F32),
            pltpu.SemaphoreType.DMA,
            pltpu.SemaphoreType.DMA,
        ],
    )
    def k(p_hbm, q_hbm, src_hbm, dst_hbm, L_hbm, s_hbm,
          pt, qt, src1, dst1, dst2, Lt, zbuf, acc_sh, st, ssm):
        c = lax.axis_index("c")
        sid = lax.axis_index("s")
        wid = c * NS + sid

        zero16 = jnp.zeros((LN,), F32)

        pltpu.async_copy(p_hbm, pt, st)
        pltpu.async_copy(q_hbm, qt, st)
        pltpu.async_copy(src_hbm.at[pl.ds(wid * ET, ET)], src1, st)
        pltpu.async_copy(dst_hbm.at[pl.ds(wid * ET, ET)], dst1, st)

        @pl.loop(0, NCH)
        def _(t):
            pltpu.async_copy(dst_hbm.at[pl.ds(wid * ET + t * 128, 128)],
                             dst2.at[t], st)

        @pl.loop(0, stripe, step=LN)
        def _(j):
            zbuf[pl.ds(j, LN)] = zero16

        pltpu.sync_copy(zbuf, acc_sh.at[pl.ds(sid * stripe, stripe)])
        pltpu.make_async_copy(p_hbm, pt, st).wait()
        pltpu.make_async_copy(q_hbm, qt, st).wait()
        pltpu.make_async_copy(src_hbm.at[pl.ds(wid * ET, ET)], src1,
                              st).wait()
        pltpu.make_async_copy(dst_hbm.at[pl.ds(wid * ET, ET)], dst1,
                              st).wait()

        @pl.loop(0, NCH)
        def _(t):
            pltpu.make_async_copy(dst_hbm.at[pl.ds(0, 128)], dst2.at[t],
                                  st).wait()

        plsc.subcore_barrier()

        @pl.loop(0, NCH)
        def _(ch):
            for g in range(8):
                base = ch * 128 + g * 16
                sv = src1[pl.ds(base, LN)]
                dv = dst1[pl.ds(base, LN)]
                pv = plsc.load_gather(pt, [sv])
                qv = plsc.load_gather(qt, [dv])
                Lt[pl.ds(base, LN)] = pv + qv
            pltpu.async_copy(Lt.at[pl.ds(ch * 128, 128)],
                             acc_sh.at[dst2.at[ch]], ssm, add=True)

        pltpu.sync_copy(Lt, L_hbm.at[pl.ds(wid * ET, ET)])

        @pl.loop(0, NCH)
        def _(ch):
            pltpu.make_async_copy(Lt.at[pl.ds(0, 128)],
                                  acc_sh.at[dst2.at[0]], ssm).wait()

        plsc.subcore_barrier()
        pltpu.sync_copy(acc_sh.at[pl.ds(sid * stripe, stripe)], zbuf)
        pltpu.sync_copy(zbuf,
                        s_hbm.at[pl.ds(c * NP + sid * stripe, stripe)])

    return k(p, q, srcp, dstp)


def _sc_final(L, srcp, dstp, s2, degc, b3b):
    EP = srcp.shape[0]
    NP = degc.shape[0]
    ET = EP // NW

    @functools.partial(
        pl.kernel,
        out_type=jax.ShapeDtypeStruct((EP,), F32),
        mesh=_mesh(),
        compiler_params=_sc_params(),
        scratch_types=[
            pltpu.VMEM((NP,), F32),
            pltpu.VMEM((NP,), F32),
            pltpu.VMEM((NP,), F32),
            pltpu.VMEM((ET,), I32),
            pltpu.VMEM((ET,), I32),
            pltpu.VMEM((ET,), F32),
            pltpu.VMEM((ET,), F32),
            pltpu.VMEM((LN,), F32),
            pltpu.SemaphoreType.DMA,
        ],
    )
    def k(L_hbm, src_hbm, dst_hbm, s_hbm, deg_hbm, b3_hbm, out_hbm,
          st, tmp, degt, src1, dst1, Lt, outt, b3t, sem):
        c = lax.axis_index("c")
        sid = lax.axis_index("s")
        wid = c * NS + sid

        copies = [
            (s_hbm.at[pl.ds(0, NP)], st),
            (s_hbm.at[pl.ds(NP, NP)], tmp),
            (deg_hbm, degt),
            (src_hbm.at[pl.ds(wid * ET, ET)], src1),
            (dst_hbm.at[pl.ds(wid * ET, ET)], dst1),
            (L_hbm.at[pl.ds(wid * ET, ET)], Lt),
            (b3_hbm, b3t),
        ]
        for s_ref, d_ref in copies:
            pltpu.async_copy(s_ref, d_ref, sem)
        for s_ref, d_ref in copies:
            pltpu.make_async_copy(s_ref, d_ref, sem).wait()

        @pl.loop(0, NP, step=LN)
        def _(j):
            st[pl.ds(j, LN)] = st[pl.ds(j, LN)] + tmp[pl.ds(j, LN)]

        b3v = b3t[...]
        one = jnp.ones((LN,), F32)
        zero = jnp.zeros((LN,), F32)

        @pl.loop(0, ET, step=LN)
        def _(j):
            sv = src1[pl.ds(j, LN)]
            dv = dst1[pl.ds(j, LN)]
            Lv = Lt[pl.ds(j, LN)]
            selfm = sv == dv
            s_at = plsc.load_gather(st, [sv])
            d_at = plsc.load_gather(degt, [sv])
            degv = jnp.maximum(d_at - jnp.where(selfm, one, zero), 1.0)
            num = s_at - jnp.where(selfm, Lv, zero)
            z = Lv + num / degv + b3v
            outt[pl.ds(j, LN)] = 1.0 / (1.0 + jnp.exp(-z))

        pltpu.sync_copy(outt, out_hbm.at[pl.ds(wid * ET, ET)])

    return k(L, srcp, dstp, s2, degc, b3b)


def kernel(x, edge_index, W1, b1, gamma1, beta1, W2, b2, W3, b3):
    N, D = x.shape
    E = edge_index.shape[1]
    NP = _round_up(N, 256)
    EP = _round_up(E, 128 * NW)

    src = edge_index[0]
    dst = edge_index[1]
    npad = EP - E
    padv = N + (jnp.arange(npad, dtype=I32) % (NP - N))
    srcp = jnp.concatenate([src, padv])
    dstp = jnp.concatenate([dst, padv])

    w3a = W3[:D, 0]
    w3b = W3[D:, 0]
    b3b = jnp.broadcast_to(b3, (LN,)).astype(F32)

    y1 = _tc_matmul_pad(x, W1, NP)
    asum1, cnt2 = _sc_rowscatter(y1, srcp, dstp, NP, with_cnt=True)
    cnt = cnt2.reshape(NC, NP)
    y2, degc = _tc_mid(y1, asum1, cnt, b1, gamma1, beta1, W2, N)
    asum2, = _sc_rowscatter(y2, srcp, dstp, NP)
    p, q = _tc_pq(y2, asum2, degc, b2, w3a, w3b)
    L, s2 = _sc_edge_logits(p, q, srcp, dstp, NP)
    outp = _sc_final(L, srcp, dstp, s2, degc, b3b)
    return outp[:E][:, None]


# speedup vs baseline: 16.8228x; 1.0102x over previous
import dataclasses
import functools

import jax
import jax.numpy as jnp
from jax import lax
from jax.experimental import pallas as pl
from jax.experimental.pallas import tpu as pltpu
from jax.experimental.pallas import tpu_sc as plsc

NC = 2
NS = 16
LN = 16
NW = NC * NS

F32 = jnp.float32
I32 = jnp.int32


def _round_up(v, m):
    return (v + m - 1) // m * m


def _mesh():
    return plsc.VectorSubcoreMesh(core_axis_name="c", subcore_axis_name="s")


def _sc_params():
    cp = pltpu.CompilerParams()
    if "needs_layout_passes" in pltpu.CompilerParams.__dataclass_fields__:
        cp = dataclasses.replace(cp, needs_layout_passes=False)
    return cp


def _tc_layer1(x, asum, cnt, W1, b1, g1, be1, W2, N):
    D = x.shape[1]

    def body(x_ref, a_ref, c_ref, w1_ref, b1_ref, g1_ref, be1_ref, w2_ref,
             h1_ref, y2_ref, deg_ref):
        deg = jnp.clip(c_ref[0, :N] + c_ref[1, :N], 1.0, None)
        asum_c = a_ref[0, :N] + a_ref[1, :N]
        y1 = jnp.dot(x_ref[...], w1_ref[...], preferred_element_type=F32)
        aggw = jnp.dot(asum_c, w1_ref[...],
                       preferred_element_type=F32) / deg[:, None]
        h = y1 + aggw + b1_ref[...][None, :]
        mu = jnp.mean(h, axis=0)
        var = jnp.mean((h - mu[None, :]) ** 2, axis=0)
        hn = (h - mu[None, :]) * lax.rsqrt(var + 1e-5)[None, :]
        hn = hn * g1_ref[...][None, :] + be1_ref[...][None, :]
        hr = jnp.maximum(hn, 0.0)
        h1_ref[...] = hr
        y2_ref[...] = jnp.dot(hr, w2_ref[...], preferred_element_type=F32)
        deg_ref[...] = deg

    return pl.pallas_call(
        body,
        out_shape=[jax.ShapeDtypeStruct((N, D), F32),
                   jax.ShapeDtypeStruct((N, D), F32),
                   jax.ShapeDtypeStruct((N,), F32)],
    )(x, asum, cnt, W1, b1, g1, be1, W2)


def _tc_pq(y2, asum, degc, b2, W2, w3a, w3b, N):
    D = y2.shape[1]

    def body(y_ref, a_ref, d_ref, b2_ref, w2_ref, wa_ref, wb_ref,
             p_ref, q_ref):
        deg = d_ref[...]
        asum_c = a_ref[0, :N] + a_ref[1, :N]
        aggw = jnp.dot(asum_c, w2_ref[...],
                       preferred_element_type=F32) / deg[:, None]
        h2 = jnp.maximum(y_ref[...] + aggw + b2_ref[...][None, :], 0.0)
        p_ref[...] = jnp.sum(h2 * wa_ref[...][None, :], axis=1)
        q_ref[...] = jnp.sum(h2 * wb_ref[...][None, :], axis=1)

    return pl.pallas_call(
        body,
        out_shape=[jax.ShapeDtypeStruct((N,), F32),
                   jax.ShapeDtypeStruct((N,), F32)],
    )(y2, asum, degc, b2, W2, w3a, w3b)


def _sc_rowscatter(y, srcp, dstp, NP, with_cnt=False):
    D = y.shape[1]
    EP = srcp.shape[0]
    NCH = EP // NW // 128
    stripe = NP // NS

    NB = 2
    out_type = [jax.ShapeDtypeStruct((NC, NP, D), F32)]
    scratch = [
        pltpu.VMEM((NCH, 128), I32),
        pltpu.VMEM((NCH, 128), I32),
    ]
    scratch += [pltpu.VMEM((128, D), F32) for _ in range(NB)]
    scratch += [
        pltpu.VMEM_SHARED((NP, D), F32),
        pltpu.SemaphoreType.DMA,
    ]
    scratch += [pltpu.SemaphoreType.DMA for _ in range(NB)]
    scratch += [pltpu.SemaphoreType.DMA for _ in range(NB)]
    if with_cnt:
        out_type.append(jax.ShapeDtypeStruct((NC * NP,), F32))
        scratch += [
            pltpu.VMEM((128,), F32),
            pltpu.VMEM((stripe,), F32),
            pltpu.VMEM_SHARED((NP,), F32),
            pltpu.SemaphoreType.DMA,
        ]

    @functools.partial(
        pl.kernel,
        out_type=out_type,
        mesh=_mesh(),
        compiler_params=_sc_params(),
        scratch_types=scratch,
    )
    def k(y_hbm, src_hbm, dst_hbm, out_hbm, *rest):
        if with_cnt:
            cnt_hbm = rest[0]
            rest = rest[1:]
        src2, dst2 = rest[0], rest[1]
        bufs = rest[2:2 + NB]
        acc_sh = rest[2 + NB]
        st = rest[3 + NB]
        gsem = rest[4 + NB:4 + 2 * NB]
        ssem = rest[4 + 2 * NB:4 + 3 * NB]
        if with_cnt:
            ones, zbuf, cnt_sh, csem = rest[4 + 3 * NB:]
        c = lax.axis_index("c")
        sid = lax.axis_index("s")
        wid = c * NS + sid
        rows0 = bufs[0]

        zero16 = jnp.zeros((LN,), F32)
        ones16 = jnp.ones((LN,), F32)

        @pl.loop(0, NCH)
        def _(t):
            base = wid * NCH * 128 + t * 128
            pltpu.async_copy(src_hbm.at[pl.ds(base, 128)], src2.at[t], st)
            pltpu.async_copy(dst_hbm.at[pl.ds(base, 128)], dst2.at[t], st)

        @pl.loop(0, 128)
        def _(r):
            for kk in range(D // LN):
                rows0[r, pl.ds(kk * LN, LN)] = zero16

        for t in range(stripe // 128):
            pltpu.sync_copy(rows0,
                            acc_sh.at[pl.ds(sid * stripe + t * 128, 128)])

        if with_cnt:
            @pl.loop(0, 128, step=LN)
            def _(j):
                ones[pl.ds(j, LN)] = ones16

            @pl.loop(0, stripe, step=LN)
            def _(j):
                zbuf[pl.ds(j, LN)] = zero16

            pltpu.sync_copy(zbuf, cnt_sh.at[pl.ds(sid * stripe, stripe)])

        @pl.loop(0, NCH)
        def _(t):
            pltpu.make_async_copy(src_hbm.at[pl.ds(0, 128)], src2.at[t],
                                  st).wait()
            pltpu.make_async_copy(dst_hbm.at[pl.ds(0, 128)], dst2.at[t],
                                  st).wait()

        plsc.subcore_barrier()

        pltpu.async_copy(y_hbm.at[src2.at[0]], bufs[0], gsem[0])
        for i in range(NCH):
            b = i % NB
            pltpu.make_async_copy(y_hbm.at[src2.at[i]], bufs[b],
                                  gsem[b]).wait()
            pltpu.async_copy(bufs[b], acc_sh.at[dst2.at[i]], ssem[b],
                             add=True)
            if with_cnt:
                pltpu.async_copy(ones, cnt_sh.at[dst2.at[i]], csem,
                                 add=True)
            if i + 1 < NCH:
                nb = (i + 1) % NB
                if i - 1 >= 0:
                    pltpu.make_async_copy(bufs[nb], acc_sh.at[dst2.at[0]],
                                          ssem[nb]).wait()
                pltpu.async_copy(y_hbm.at[src2.at[i + 1]], bufs[nb],
                                 gsem[nb])
        for i in range(NCH - 2, NCH):
            pltpu.make_async_copy(bufs[i % NB], acc_sh.at[dst2.at[0]],
                                  ssem[i % NB]).wait()
        if with_cnt:
            for i in range(NCH):
                pltpu.make_async_copy(ones, cnt_sh.at[dst2.at[0]],
                                      csem).wait()

        plsc.subcore_barrier()
        for t in range(stripe // 128):
            pltpu.sync_copy(acc_sh.at[pl.ds(sid * stripe + t * 128, 128)],
                            rows0)
            pltpu.sync_copy(rows0,
                            out_hbm.at[c, pl.ds(sid * stripe + t * 128, 128)])
        if with_cnt:
            pltpu.sync_copy(cnt_sh.at[pl.ds(sid * stripe, stripe)], zbuf)
            pltpu.sync_copy(zbuf,
                            cnt_hbm.at[pl.ds(c * NP + sid * stripe, stripe)])

    return k(y, srcp, dstp)


def _sc_edge_logits(p, q, srcp, dstp, NP):
    EP = srcp.shape[0]
    NN = p.shape[0]
    ET = EP // NW
    NCH = ET // 128
    stripe = NP // NS

    @functools.partial(
        pl.kernel,
        out_type=[jax.ShapeDtypeStruct((EP,), F32),
                  jax.ShapeDtypeStruct((NC * NP,), F32)],
        mesh=_mesh(),
        compiler_params=_sc_params(),
        scratch_types=[
            pltpu.VMEM((NN,), F32),
            pltpu.VMEM((NN,), F32),
            pltpu.VMEM((ET,), I32),
            pltpu.VMEM((ET,), I32),
            pltpu.VMEM((NCH, 128), I32),
            pltpu.VMEM((ET,), F32),
            pltpu.VMEM((stripe,), F32),
            pltpu.VMEM_SHARED((NP,), F32),
            pltpu.SemaphoreType.DMA,
            pltpu.SemaphoreType.DMA,
        ],
    )
    def k(p_hbm, q_hbm, src_hbm, dst_hbm, L_hbm, s_hbm,
          pt, qt, src1, dst1, dst2, Lt, zbuf, acc_sh, st, ssm):
        c = lax.axis_index("c")
        sid = lax.axis_index("s")
        wid = c * NS + sid

        zero16 = jnp.zeros((LN,), F32)

        pltpu.async_copy(p_hbm, pt, st)
        pltpu.async_copy(q_hbm, qt, st)
        pltpu.async_copy(src_hbm.at[pl.ds(wid * ET, ET)], src1, st)
        pltpu.async_copy(dst_hbm.at[pl.ds(wid * ET, ET)], dst1, st)

        @pl.loop(0, NCH)
        def _(t):
            pltpu.async_copy(dst_hbm.at[pl.ds(wid * ET + t * 128, 128)],
                             dst2.at[t], st)

        @pl.loop(0, stripe, step=LN)
        def _(j):
            zbuf[pl.ds(j, LN)] = zero16

        pltpu.sync_copy(zbuf, acc_sh.at[pl.ds(sid * stripe, stripe)])
        pltpu.make_async_copy(p_hbm, pt, st).wait()
        pltpu.make_async_copy(q_hbm, qt, st).wait()
        pltpu.make_async_copy(src_hbm.at[pl.ds(wid * ET, ET)], src1,
                              st).wait()
        pltpu.make_async_copy(dst_hbm.at[pl.ds(wid * ET, ET)], dst1,
                              st).wait()

        @pl.loop(0, NCH)
        def _(t):
            pltpu.make_async_copy(dst_hbm.at[pl.ds(0, 128)], dst2.at[t],
                                  st).wait()

        plsc.subcore_barrier()

        @pl.loop(0, NCH)
        def _(ch):
            for g in range(8):
                base = ch * 128 + g * 16
                sv = src1[pl.ds(base, LN)]
                dv = dst1[pl.ds(base, LN)]
                pv = plsc.load_gather(pt, [sv])
                qv = plsc.load_gather(qt, [dv])
                Lt[pl.ds(base, LN)] = pv + qv
            pltpu.async_copy(Lt.at[pl.ds(ch * 128, 128)],
                             acc_sh.at[dst2.at[ch]], ssm, add=True)

        pltpu.sync_copy(Lt, L_hbm.at[pl.ds(wid * ET, ET)])

        @pl.loop(0, NCH)
        def _(ch):
            pltpu.make_async_copy(Lt.at[pl.ds(0, 128)],
                                  acc_sh.at[dst2.at[0]], ssm).wait()

        plsc.subcore_barrier()
        pltpu.sync_copy(acc_sh.at[pl.ds(sid * stripe, stripe)], zbuf)
        pltpu.sync_copy(zbuf,
                        s_hbm.at[pl.ds(c * NP + sid * stripe, stripe)])

    return k(p, q, srcp, dstp)


def _sc_final(L, srcp, dstp, s2, degc, b3b):
    EP = srcp.shape[0]
    NN = degc.shape[0]
    NP = s2.shape[0] // NC
    ET = EP // NW

    @functools.partial(
        pl.kernel,
        out_type=jax.ShapeDtypeStruct((EP,), F32),
        mesh=_mesh(),
        compiler_params=_sc_params(),
        scratch_types=[
            pltpu.VMEM((NN,), F32),
            pltpu.VMEM((NN,), F32),
            pltpu.VMEM((NN,), F32),
            pltpu.VMEM((ET,), I32),
            pltpu.VMEM((ET,), I32),
            pltpu.VMEM((ET,), F32),
            pltpu.VMEM((ET,), F32),
            pltpu.VMEM((LN,), F32),
            pltpu.SemaphoreType.DMA,
        ],
    )
    def k(L_hbm, src_hbm, dst_hbm, s_hbm, deg_hbm, b3_hbm, out_hbm,
          st, tmp, degt, src1, dst1, Lt, outt, b3t, sem):
        c = lax.axis_index("c")
        sid = lax.axis_index("s")
        wid = c * NS + sid

        copies = [
            (s_hbm.at[pl.ds(0, NN)], st),
            (s_hbm.at[pl.ds(NP, NN)], tmp),
            (deg_hbm, degt),
            (src_hbm.at[pl.ds(wid * ET, ET)], src1),
            (dst_hbm.at[pl.ds(wid * ET, ET)], dst1),
            (L_hbm.at[pl.ds(wid * ET, ET)], Lt),
            (b3_hbm, b3t),
        ]
        for s_ref, d_ref in copies:
            pltpu.async_copy(s_ref, d_ref, sem)
        for s_ref, d_ref in copies:
            pltpu.make_async_copy(s_ref, d_ref, sem).wait()

        @pl.loop(0, NN, step=LN)
        def _(j):
            st[pl.ds(j, LN)] = st[pl.ds(j, LN)] + tmp[pl.ds(j, LN)]

        b3v = b3t[...]
        one = jnp.ones((LN,), F32)
        zero = jnp.zeros((LN,), F32)

        @pl.loop(0, ET, step=LN)
        def _(j):
            sv = src1[pl.ds(j, LN)]
            dv = dst1[pl.ds(j, LN)]
            Lv = Lt[pl.ds(j, LN)]
            selfm = sv == dv
            s_at = plsc.load_gather(st, [sv])
            d_at = plsc.load_gather(degt, [sv])
            degv = jnp.maximum(d_at - jnp.where(selfm, one, zero), 1.0)
            num = s_at - jnp.where(selfm, Lv, zero)
            z = Lv + num / degv + b3v
            outt[pl.ds(j, LN)] = 1.0 / (1.0 + jnp.exp(-z))

        pltpu.sync_copy(outt, out_hbm.at[pl.ds(wid * ET, ET)])

    return k(L, srcp, dstp, s2, degc, b3b)


def kernel(x, edge_index, W1, b1, gamma1, beta1, W2, b2, W3, b3):
    N, D = x.shape
    E = edge_index.shape[1]
    NP = _round_up(N, 256)
    EP = _round_up(E, 128 * NW)

    src = edge_index[0]
    dst = edge_index[1]
    npad = EP - E
    padd = N + (jnp.arange(npad, dtype=I32) % (NP - N))
    pads = jnp.arange(npad, dtype=I32) % (NP - N)
    srcp = jnp.concatenate([src, pads])
    dstp = jnp.concatenate([dst, padd])

    w3a = W3[:D, 0]
    w3b = W3[D:, 0]
    b3b = jnp.broadcast_to(b3, (LN,)).astype(F32)

    asum1, cnt2 = _sc_rowscatter(x, srcp, dstp, NP, with_cnt=True)
    cnt = cnt2.reshape(NC, NP)
    h1, y2, degc = _tc_layer1(x, asum1, cnt, W1, b1, gamma1, beta1, W2, N)
    asum2, = _sc_rowscatter(h1, srcp, dstp, NP)
    p, q = _tc_pq(y2, asum2, degc, b2, W2, w3a, w3b, N)
    L, s2 = _sc_edge_logits(p, q, srcp, dstp, NP)
    outp = _sc_final(L, srcp, dstp, s2, degc, b3b)
    return outp[:E][:, None]


# speedup vs baseline: 16.8798x; 1.0034x over previous
import dataclasses
import functools

import jax
import jax.numpy as jnp
from jax import lax
from jax.experimental import pallas as pl
from jax.experimental.pallas import tpu as pltpu
from jax.experimental.pallas import tpu_sc as plsc

NC = 2
NS = 16
LN = 16
NW = NC * NS

F32 = jnp.float32
I32 = jnp.int32


def _round_up(v, m):
    return (v + m - 1) // m * m


def _mesh():
    return plsc.VectorSubcoreMesh(core_axis_name="c", subcore_axis_name="s")


def _sc_params():
    cp = pltpu.CompilerParams()
    if "needs_layout_passes" in pltpu.CompilerParams.__dataclass_fields__:
        cp = dataclasses.replace(cp, needs_layout_passes=False)
    return cp


def _tc_layer1(x, asum, cnt, W1, b1, g1, be1, W2, N):
    D = x.shape[1]

    def body(x_ref, a_ref, c_ref, w1_ref, b1_ref, g1_ref, be1_ref, w2_ref,
             h1_ref, y2_ref, deg_ref):
        deg = jnp.clip(c_ref[0, :N] + c_ref[1, :N], 1.0, None)
        asum_c = a_ref[0, :N] + a_ref[1, :N]
        y1 = jnp.dot(x_ref[...], w1_ref[...], preferred_element_type=F32)
        aggw = jnp.dot(asum_c, w1_ref[...],
                       preferred_element_type=F32) / deg[:, None]
        h = y1 + aggw + b1_ref[...][None, :]
        mu = jnp.mean(h, axis=0)
        var = jnp.mean((h - mu[None, :]) ** 2, axis=0)
        hn = (h - mu[None, :]) * lax.rsqrt(var + 1e-5)[None, :]
        hn = hn * g1_ref[...][None, :] + be1_ref[...][None, :]
        hr = jnp.maximum(hn, 0.0)
        h1_ref[...] = hr
        y2_ref[...] = jnp.dot(hr, w2_ref[...], preferred_element_type=F32)
        deg_ref[...] = deg

    return pl.pallas_call(
        body,
        out_shape=[jax.ShapeDtypeStruct((N, D), F32),
                   jax.ShapeDtypeStruct((N, D), F32),
                   jax.ShapeDtypeStruct((N,), F32)],
    )(x, asum, cnt, W1, b1, g1, be1, W2)


def _tc_pq(y2, asum, degc, b2, W2, w3a, w3b, N):
    D = y2.shape[1]

    def body(y_ref, a_ref, d_ref, b2_ref, w2_ref, wa_ref, wb_ref,
             p_ref, q_ref, r0_ref, r1_ref):
        deg = d_ref[...]
        asum_c = a_ref[0, :N] + a_ref[1, :N]
        aggw = jnp.dot(asum_c, w2_ref[...],
                       preferred_element_type=F32) / deg[:, None]
        h2 = jnp.maximum(y_ref[...] + aggw + b2_ref[...][None, :], 0.0)
        p_ref[...] = jnp.sum(h2 * wa_ref[...][None, :], axis=1)
        q_ref[...] = jnp.sum(h2 * wb_ref[...][None, :], axis=1)
        r0_ref[...] = 1.0 / deg
        r1_ref[...] = 1.0 / jnp.maximum(deg - 1.0, 1.0)

    return pl.pallas_call(
        body,
        out_shape=[jax.ShapeDtypeStruct((N,), F32),
                   jax.ShapeDtypeStruct((N,), F32),
                   jax.ShapeDtypeStruct((N,), F32),
                   jax.ShapeDtypeStruct((N,), F32)],
    )(y2, asum, degc, b2, W2, w3a, w3b)


def _sc_rowscatter(y, srcp, dstp, NP, with_cnt=False):
    D = y.shape[1]
    EP = srcp.shape[0]
    NCH = EP // NW // 128
    stripe = NP // NS

    NB = 2
    out_type = [jax.ShapeDtypeStruct((NC, NP, D), F32)]
    scratch = [
        pltpu.VMEM((NCH, 128), I32),
        pltpu.VMEM((NCH, 128), I32),
    ]
    scratch += [pltpu.VMEM((128, D), F32) for _ in range(NB)]
    scratch += [
        pltpu.VMEM_SHARED((NP, D), F32),
        pltpu.SemaphoreType.DMA,
    ]
    scratch += [pltpu.SemaphoreType.DMA for _ in range(NB)]
    scratch += [pltpu.SemaphoreType.DMA for _ in range(NB)]
    if with_cnt:
        out_type.append(jax.ShapeDtypeStruct((NC * NP,), F32))
        scratch += [
            pltpu.VMEM((128,), F32),
            pltpu.VMEM((stripe,), F32),
            pltpu.VMEM_SHARED((NP,), F32),
            pltpu.SemaphoreType.DMA,
        ]

    @functools.partial(
        pl.kernel,
        out_type=out_type,
        mesh=_mesh(),
        compiler_params=_sc_params(),
        scratch_types=scratch,
    )
    def k(y_hbm, src_hbm, dst_hbm, out_hbm, *rest):
        if with_cnt:
            cnt_hbm = rest[0]
            rest = rest[1:]
        src2, dst2 = rest[0], rest[1]
        bufs = rest[2:2 + NB]
        acc_sh = rest[2 + NB]
        st = rest[3 + NB]
        gsem = rest[4 + NB:4 + 2 * NB]
        ssem = rest[4 + 2 * NB:4 + 3 * NB]
        if with_cnt:
            ones, zbuf, cnt_sh, csem = rest[4 + 3 * NB:]
        c = lax.axis_index("c")
        sid = lax.axis_index("s")
        wid = c * NS + sid
        rows0 = bufs[0]

        zero16 = jnp.zeros((LN,), F32)
        ones16 = jnp.ones((LN,), F32)

        @pl.loop(0, NCH)
        def _(t):
            base = wid * NCH * 128 + t * 128
            pltpu.async_copy(src_hbm.at[pl.ds(base, 128)], src2.at[t], st)
            pltpu.async_copy(dst_hbm.at[pl.ds(base, 128)], dst2.at[t], st)

        @pl.loop(0, 128)
        def _(r):
            for kk in range(D // LN):
                rows0[r, pl.ds(kk * LN, LN)] = zero16

        for t in range(stripe // 128):
            pltpu.sync_copy(rows0,
                            acc_sh.at[pl.ds(sid * stripe + t * 128, 128)])

        if with_cnt:
            @pl.loop(0, 128, step=LN)
            def _(j):
                ones[pl.ds(j, LN)] = ones16

            @pl.loop(0, stripe, step=LN)
            def _(j):
                zbuf[pl.ds(j, LN)] = zero16

            pltpu.sync_copy(zbuf, cnt_sh.at[pl.ds(sid * stripe, stripe)])

        @pl.loop(0, NCH)
        def _(t):
            pltpu.make_async_copy(src_hbm.at[pl.ds(0, 128)], src2.at[t],
                                  st).wait()
            pltpu.make_async_copy(dst_hbm.at[pl.ds(0, 128)], dst2.at[t],
                                  st).wait()

        plsc.subcore_barrier()

        pltpu.async_copy(y_hbm.at[src2.at[0]], bufs[0], gsem[0])
        for i in range(NCH):
            b = i % NB
            pltpu.make_async_copy(y_hbm.at[src2.at[i]], bufs[b],
                                  gsem[b]).wait()
            pltpu.async_copy(bufs[b], acc_sh.at[dst2.at[i]], ssem[b],
                             add=True)
            if with_cnt:
                pltpu.async_copy(ones, cnt_sh.at[dst2.at[i]], csem,
                                 add=True)
            if i + 1 < NCH:
                nb = (i + 1) % NB
                if i - 1 >= 0:
                    pltpu.make_async_copy(bufs[nb], acc_sh.at[dst2.at[0]],
                                          ssem[nb]).wait()
                pltpu.async_copy(y_hbm.at[src2.at[i + 1]], bufs[nb],
                                 gsem[nb])
        for i in range(NCH - 2, NCH):
            pltpu.make_async_copy(bufs[i % NB], acc_sh.at[dst2.at[0]],
                                  ssem[i % NB]).wait()
        if with_cnt:
            for i in range(NCH):
                pltpu.make_async_copy(ones, cnt_sh.at[dst2.at[0]],
                                      csem).wait()

        plsc.subcore_barrier()
        for t in range(stripe // 128):
            pltpu.sync_copy(acc_sh.at[pl.ds(sid * stripe + t * 128, 128)],
                            rows0)
            pltpu.sync_copy(rows0,
                            out_hbm.at[c, pl.ds(sid * stripe + t * 128, 128)])
        if with_cnt:
            pltpu.sync_copy(cnt_sh.at[pl.ds(sid * stripe, stripe)], zbuf)
            pltpu.sync_copy(zbuf,
                            cnt_hbm.at[pl.ds(c * NP + sid * stripe, stripe)])

    return k(y, srcp, dstp)


def _sc_edge_logits(p, q, srcp, dstp, NP):
    EP = srcp.shape[0]
    NN = p.shape[0]
    ET = EP // NW
    NCH = ET // 128
    stripe = NP // NS

    @functools.partial(
        pl.kernel,
        out_type=[jax.ShapeDtypeStruct((EP,), F32),
                  jax.ShapeDtypeStruct((NC * NP,), F32)],
        mesh=_mesh(),
        compiler_params=_sc_params(),
        scratch_types=[
            pltpu.VMEM((NN,), F32),
            pltpu.VMEM((NN,), F32),
            pltpu.VMEM((ET,), I32),
            pltpu.VMEM((ET,), I32),
            pltpu.VMEM((NCH, 128), I32),
            pltpu.VMEM((ET,), F32),
            pltpu.VMEM((stripe,), F32),
            pltpu.VMEM_SHARED((NP,), F32),
            pltpu.SemaphoreType.DMA,
            pltpu.SemaphoreType.DMA,
        ],
    )
    def k(p_hbm, q_hbm, src_hbm, dst_hbm, L_hbm, s_hbm,
          pt, qt, src1, dst1, dst2, Lt, zbuf, acc_sh, st, ssm):
        c = lax.axis_index("c")
        sid = lax.axis_index("s")
        wid = c * NS + sid

        zero16 = jnp.zeros((LN,), F32)

        pltpu.async_copy(p_hbm, pt, st)
        pltpu.async_copy(q_hbm, qt, st)
        pltpu.async_copy(src_hbm.at[pl.ds(wid * ET, ET)], src1, st)
        pltpu.async_copy(dst_hbm.at[pl.ds(wid * ET, ET)], dst1, st)

        @pl.loop(0, NCH)
        def _(t):
            pltpu.async_copy(dst_hbm.at[pl.ds(wid * ET + t * 128, 128)],
                             dst2.at[t], st)

        @pl.loop(0, stripe, step=LN)
        def _(j):
            zbuf[pl.ds(j, LN)] = zero16

        pltpu.sync_copy(zbuf, acc_sh.at[pl.ds(sid * stripe, stripe)])
        pltpu.make_async_copy(p_hbm, pt, st).wait()
        pltpu.make_async_copy(q_hbm, qt, st).wait()
        pltpu.make_async_copy(src_hbm.at[pl.ds(wid * ET, ET)], src1,
                              st).wait()
        pltpu.make_async_copy(dst_hbm.at[pl.ds(wid * ET, ET)], dst1,
                              st).wait()

        @pl.loop(0, NCH)
        def _(t):
            pltpu.make_async_copy(dst_hbm.at[pl.ds(0, 128)], dst2.at[t],
                                  st).wait()

        plsc.subcore_barrier()

        @pl.loop(0, NCH)
        def _(ch):
            for g in range(8):
                base = ch * 128 + g * 16
                sv = src1[pl.ds(base, LN)]
                dv = dst1[pl.ds(base, LN)]
                pv = plsc.load_gather(pt, [sv])
                qv = plsc.load_gather(qt, [dv])
                Lt[pl.ds(base, LN)] = pv + qv
            pltpu.async_copy(Lt.at[pl.ds(ch * 128, 128)],
                             acc_sh.at[dst2.at[ch]], ssm, add=True)

        pltpu.sync_copy(Lt, L_hbm.at[pl.ds(wid * ET, ET)])

        @pl.loop(0, NCH)
        def _(ch):
            pltpu.make_async_copy(Lt.at[pl.ds(0, 128)],
                                  acc_sh.at[dst2.at[0]], ssm).wait()

        plsc.subcore_barrier()
        pltpu.sync_copy(acc_sh.at[pl.ds(sid * stripe, stripe)], zbuf)
        pltpu.sync_copy(zbuf,
                        s_hbm.at[pl.ds(c * NP + sid * stripe, stripe)])

    return k(p, q, srcp, dstp)


def _sc_final(L, srcp, dstp, s2, r0, r1, b3b):
    EP = srcp.shape[0]
    NN = r0.shape[0]
    NP = s2.shape[0] // NC
    ET = EP // NW

    @functools.partial(
        pl.kernel,
        out_type=jax.ShapeDtypeStruct((EP,), F32),
        mesh=_mesh(),
        compiler_params=_sc_params(),
        scratch_types=[
            pltpu.VMEM((NN,), F32),
            pltpu.VMEM((NN,), F32),
            pltpu.VMEM((NN,), F32),
            pltpu.VMEM((NN,), F32),
            pltpu.VMEM((ET,), I32),
            pltpu.VMEM((ET,), I32),
            pltpu.VMEM((ET,), F32),
            pltpu.VMEM((ET,), F32),
            pltpu.VMEM((LN,), F32),
            pltpu.SemaphoreType.DMA,
        ],
    )
    def k(L_hbm, src_hbm, dst_hbm, s_hbm, r0_hbm, r1_hbm, b3_hbm, out_hbm,
          st, tmp, r0t, r1t, src1, dst1, Lt, outt, b3t, sem):
        c = lax.axis_index("c")
        sid = lax.axis_index("s")
        wid = c * NS + sid

        copies = [
            (s_hbm.at[pl.ds(0, NN)], st),
            (s_hbm.at[pl.ds(NP, NN)], tmp),
            (r0_hbm, r0t),
            (r1_hbm, r1t),
            (src_hbm.at[pl.ds(wid * ET, ET)], src1),
            (dst_hbm.at[pl.ds(wid * ET, ET)], dst1),
            (L_hbm.at[pl.ds(wid * ET, ET)], Lt),
            (b3_hbm, b3t),
        ]
        for s_ref, d_ref in copies:
            pltpu.async_copy(s_ref, d_ref, sem)
        for s_ref, d_ref in copies:
            pltpu.make_async_copy(s_ref, d_ref, sem).wait()

        @pl.loop(0, NN, step=LN)
        def _(j):
            st[pl.ds(j, LN)] = st[pl.ds(j, LN)] + tmp[pl.ds(j, LN)]

        b3v = b3t[...]
        one = jnp.ones((LN,), F32)
        zero = jnp.zeros((LN,), F32)

        @pl.loop(0, ET, step=LN)
        def _(j):
            sv = src1[pl.ds(j, LN)]
            dv = dst1[pl.ds(j, LN)]
            Lv = Lt[pl.ds(j, LN)]
            selfm = sv == dv
            s_at = plsc.load_gather(st, [sv])
            r0v = plsc.load_gather(r0t, [sv])
            r1v = plsc.load_gather(r1t, [sv])
            rv = jnp.where(selfm, r1v, r0v)
            num = s_at - jnp.where(selfm, Lv, zero)
            z = Lv + num * rv + b3v
            outt[pl.ds(j, LN)] = 1.0 / (1.0 + jnp.exp(-z))

        pltpu.sync_copy(outt, out_hbm.at[pl.ds(wid * ET, ET)])

    return k(L, srcp, dstp, s2, r0, r1, b3b)


def kernel(x, edge_index, W1, b1, gamma1, beta1, W2, b2, W3, b3):
    N, D = x.shape
    E = edge_index.shape[1]
    NP = _round_up(N, 256)
    EP = _round_up(E, 128 * NW)

    src = edge_index[0]
    dst = edge_index[1]
    npad = EP - E
    padd = N + (jnp.arange(npad, dtype=I32) % (NP - N))
    pads = jnp.arange(npad, dtype=I32) % (NP - N)
    srcp = jnp.concatenate([src, pads])
    dstp = jnp.concatenate([dst, padd])

    w3a = W3[:D, 0]
    w3b = W3[D:, 0]
    b3b = jnp.broadcast_to(b3, (LN,)).astype(F32)

    asum1, cnt2 = _sc_rowscatter(x, srcp, dstp, NP, with_cnt=True)
    cnt = cnt2.reshape(NC, NP)
    h1, y2, degc = _tc_layer1(x, asum1, cnt, W1, b1, gamma1, beta1, W2, N)
    asum2, = _sc_rowscatter(h1, srcp, dstp, NP)
    p, q, r0, r1 = _tc_pq(y2, asum2, degc, b2, W2, w3a, w3b, N)
    L, s2 = _sc_edge_logits(p, q, srcp, dstp, NP)
    outp = _sc_final(L, srcp, dstp, s2, r0, r1, b3b)
    return outp[:E][:, None]


# speedup vs baseline: 18.2964x; 1.0839x over previous
import dataclasses
import functools

import jax
import jax.numpy as jnp
from jax import lax
from jax.experimental import pallas as pl
from jax.experimental.pallas import tpu as pltpu
from jax.experimental.pallas import tpu_sc as plsc

NC = 2
NS = 16
LN = 16
NW = NC * NS

F32 = jnp.float32
I32 = jnp.int32


def _round_up(v, m):
    return (v + m - 1) // m * m


def _mesh():
    return plsc.VectorSubcoreMesh(core_axis_name="c", subcore_axis_name="s")


def _sc_params():
    cp = pltpu.CompilerParams()
    if "needs_layout_passes" in pltpu.CompilerParams.__dataclass_fields__:
        cp = dataclasses.replace(cp, needs_layout_passes=False)
    return cp


def _tc_layer1(x, asum, cnt, W1, b1, g1, be1, W2, N):
    D = x.shape[1]

    def body(x_ref, a_ref, c_ref, w1_ref, b1_ref, g1_ref, be1_ref, w2_ref,
             h1_ref, y2_ref, deg_ref):
        deg = jnp.clip(c_ref[0, :N] + c_ref[1, :N], 1.0, None)
        asum_c = a_ref[0, :N] + a_ref[1, :N]
        y1 = jnp.dot(x_ref[...], w1_ref[...], preferred_element_type=F32)
        aggw = jnp.dot(asum_c, w1_ref[...],
                       preferred_element_type=F32) / deg[:, None]
        h = y1 + aggw + b1_ref[...][None, :]
        mu = jnp.mean(h, axis=0)
        var = jnp.mean((h - mu[None, :]) ** 2, axis=0)
        hn = (h - mu[None, :]) * lax.rsqrt(var + 1e-5)[None, :]
        hn = hn * g1_ref[...][None, :] + be1_ref[...][None, :]
        hr = jnp.maximum(hn, 0.0)
        h1_ref[...] = hr
        y2_ref[...] = jnp.dot(hr, w2_ref[...], preferred_element_type=F32)
        deg_ref[...] = deg

    return pl.pallas_call(
        body,
        out_shape=[jax.ShapeDtypeStruct((N, D), F32),
                   jax.ShapeDtypeStruct((N, D), F32),
                   jax.ShapeDtypeStruct((N,), F32)],
    )(x, asum, cnt, W1, b1, g1, be1, W2)


def _tc_pq(y2, asum, degc, b2, W2, w3a, w3b, N):
    D = y2.shape[1]

    def body(y_ref, a_ref, d_ref, b2_ref, w2_ref, wa_ref, wb_ref,
             p_ref, q_ref, r0_ref, r1_ref):
        deg = d_ref[...]
        asum_c = a_ref[0, :N] + a_ref[1, :N]
        aggw = jnp.dot(asum_c, w2_ref[...],
                       preferred_element_type=F32) / deg[:, None]
        h2 = jnp.maximum(y_ref[...] + aggw + b2_ref[...][None, :], 0.0)
        p_ref[...] = jnp.sum(h2 * wa_ref[...][None, :], axis=1)
        q_ref[...] = jnp.sum(h2 * wb_ref[...][None, :], axis=1)
        r0_ref[...] = 1.0 / deg
        r1_ref[...] = 1.0 / jnp.maximum(deg - 1.0, 1.0)

    return pl.pallas_call(
        body,
        out_shape=[jax.ShapeDtypeStruct((N,), F32),
                   jax.ShapeDtypeStruct((N,), F32),
                   jax.ShapeDtypeStruct((N,), F32),
                   jax.ShapeDtypeStruct((N,), F32)],
    )(y2, asum, degc, b2, W2, w3a, w3b)


def _sc_rowscatter(y, srcp, dstp, NP, with_cnt=False):
    D = y.shape[1]
    EP = srcp.shape[0]
    CH = 64
    NCH = EP // NW // CH
    stripe = NP // NS

    NB = 3
    out_type = [jax.ShapeDtypeStruct((NC, NP, D), F32)]
    scratch = [
        pltpu.VMEM((NCH, CH), I32),
        pltpu.VMEM((NCH, CH), I32),
    ]
    scratch += [pltpu.VMEM((CH, D), F32) for _ in range(NB)]
    scratch += [
        pltpu.VMEM_SHARED((NP, D), F32),
        pltpu.SemaphoreType.DMA,
    ]
    scratch += [pltpu.SemaphoreType.DMA for _ in range(NB)]
    scratch += [pltpu.SemaphoreType.DMA for _ in range(NB)]
    if with_cnt:
        out_type.append(jax.ShapeDtypeStruct((NC * NP,), F32))
        scratch += [
            pltpu.VMEM((CH,), F32),
            pltpu.VMEM((stripe,), F32),
            pltpu.VMEM_SHARED((NP,), F32),
            pltpu.SemaphoreType.DMA,
        ]

    @functools.partial(
        pl.kernel,
        out_type=out_type,
        mesh=_mesh(),
        compiler_params=_sc_params(),
        scratch_types=scratch,
    )
    def k(y_hbm, src_hbm, dst_hbm, out_hbm, *rest):
        if with_cnt:
            cnt_hbm = rest[0]
            rest = rest[1:]
        src2, dst2 = rest[0], rest[1]
        bufs = rest[2:2 + NB]
        acc_sh = rest[2 + NB]
        st = rest[3 + NB]
        gsem = rest[4 + NB:4 + 2 * NB]
        ssem = rest[4 + 2 * NB:4 + 3 * NB]
        if with_cnt:
            ones, zbuf, cnt_sh, csem = rest[4 + 3 * NB:]
        c = lax.axis_index("c")
        sid = lax.axis_index("s")
        wid = c * NS + sid
        rows0 = bufs[0]

        zero16 = jnp.zeros((LN,), F32)
        ones16 = jnp.ones((LN,), F32)

        @pl.loop(0, NCH)
        def _(t):
            base = wid * NCH * CH + t * CH
            pltpu.async_copy(src_hbm.at[pl.ds(base, CH)], src2.at[t], st)
            pltpu.async_copy(dst_hbm.at[pl.ds(base, CH)], dst2.at[t], st)

        @pl.loop(0, CH)
        def _(r):
            for kk in range(D // LN):
                rows0[r, pl.ds(kk * LN, LN)] = zero16

        for t in range(stripe // CH):
            pltpu.sync_copy(rows0,
                            acc_sh.at[pl.ds(sid * stripe + t * CH, CH)])

        if with_cnt:
            @pl.loop(0, CH, step=LN)
            def _(j):
                ones[pl.ds(j, LN)] = ones16

            @pl.loop(0, stripe, step=LN)
            def _(j):
                zbuf[pl.ds(j, LN)] = zero16

            pltpu.sync_copy(zbuf, cnt_sh.at[pl.ds(sid * stripe, stripe)])

        @pl.loop(0, NCH)
        def _(t):
            pltpu.make_async_copy(src_hbm.at[pl.ds(0, CH)], src2.at[t],
                                  st).wait()
            pltpu.make_async_copy(dst_hbm.at[pl.ds(0, CH)], dst2.at[t],
                                  st).wait()

        plsc.subcore_barrier()

        pltpu.async_copy(y_hbm.at[src2.at[0]], bufs[0], gsem[0])
        pltpu.async_copy(y_hbm.at[src2.at[1]], bufs[1], gsem[1])
        for i in range(NCH):
            b = i % NB
            pltpu.make_async_copy(y_hbm.at[src2.at[i]], bufs[b],
                                  gsem[b]).wait()
            pltpu.async_copy(bufs[b], acc_sh.at[dst2.at[i]], ssem[b],
                             add=True)
            if with_cnt:
                pltpu.async_copy(ones, cnt_sh.at[dst2.at[i]], csem,
                                 add=True)
            if i + 2 < NCH:
                nb = (i + 2) % NB
                if i + 2 - NB >= 0:
                    pltpu.make_async_copy(bufs[nb], acc_sh.at[dst2.at[0]],
                                          ssem[nb]).wait()
                pltpu.async_copy(y_hbm.at[src2.at[i + 2]], bufs[nb],
                                 gsem[nb])
        for i in range(max(NCH - NB, 0), NCH):
            pltpu.make_async_copy(bufs[i % NB], acc_sh.at[dst2.at[0]],
                                  ssem[i % NB]).wait()
        if with_cnt:
            for i in range(NCH):
                pltpu.make_async_copy(ones, cnt_sh.at[dst2.at[0]],
                                      csem).wait()

        plsc.subcore_barrier()
        for t in range(stripe // CH):
            pltpu.sync_copy(acc_sh.at[pl.ds(sid * stripe + t * CH, CH)],
                            rows0)
            pltpu.sync_copy(rows0,
                            out_hbm.at[c, pl.ds(sid * stripe + t * CH, CH)])
        if with_cnt:
            pltpu.sync_copy(cnt_sh.at[pl.ds(sid * stripe, stripe)], zbuf)
            pltpu.sync_copy(zbuf,
                            cnt_hbm.at[pl.ds(c * NP + sid * stripe, stripe)])

    return k(y, srcp, dstp)


def _sc_edge_logits(p, q, srcp, dstp, NP):
    EP = srcp.shape[0]
    NN = p.shape[0]
    ET = EP // NW
    NCH = ET // 128
    stripe = NP // NS

    @functools.partial(
        pl.kernel,
        out_type=[jax.ShapeDtypeStruct((EP,), F32),
                  jax.ShapeDtypeStruct((NC * NP,), F32)],
        mesh=_mesh(),
        compiler_params=_sc_params(),
        scratch_types=[
            pltpu.VMEM((NN,), F32),
            pltpu.VMEM((NN,), F32),
            pltpu.VMEM((ET,), I32),
            pltpu.VMEM((ET,), I32),
            pltpu.VMEM((NCH, 128), I32),
            pltpu.VMEM((ET,), F32),
            pltpu.VMEM((stripe,), F32),
            pltpu.VMEM_SHARED((NP,), F32),
            pltpu.SemaphoreType.DMA,
            pltpu.SemaphoreType.DMA,
        ],
    )
    def k(p_hbm, q_hbm, src_hbm, dst_hbm, L_hbm, s_hbm,
          pt, qt, src1, dst1, dst2, Lt, zbuf, acc_sh, st, ssm):
        c = lax.axis_index("c")
        sid = lax.axis_index("s")
        wid = c * NS + sid

        zero16 = jnp.zeros((LN,), F32)

        pltpu.async_copy(p_hbm, pt, st)
        pltpu.async_copy(q_hbm, qt, st)
        pltpu.async_copy(src_hbm.at[pl.ds(wid * ET, ET)], src1, st)
        pltpu.async_copy(dst_hbm.at[pl.ds(wid * ET, ET)], dst1, st)

        @pl.loop(0, NCH)
        def _(t):
            pltpu.async_copy(dst_hbm.at[pl.ds(wid * ET + t * 128, 128)],
                             dst2.at[t], st)

        @pl.loop(0, stripe, step=LN)
        def _(j):
            zbuf[pl.ds(j, LN)] = zero16

        pltpu.sync_copy(zbuf, acc_sh.at[pl.ds(sid * stripe, stripe)])
        pltpu.make_async_copy(p_hbm, pt, st).wait()
        pltpu.make_async_copy(q_hbm, qt, st).wait()
        pltpu.make_async_copy(src_hbm.at[pl.ds(wid * ET, ET)], src1,
                              st).wait()
        pltpu.make_async_copy(dst_hbm.at[pl.ds(wid * ET, ET)], dst1,
                              st).wait()

        @pl.loop(0, NCH)
        def _(t):
            pltpu.make_async_copy(dst_hbm.at[pl.ds(0, 128)], dst2.at[t],
                                  st).wait()

        plsc.subcore_barrier()

        @pl.loop(0, NCH)
        def _(ch):
            for g in range(8):
                base = ch * 128 + g * 16
                sv = src1[pl.ds(base, LN)]
                dv = dst1[pl.ds(base, LN)]
                pv = plsc.load_gather(pt, [sv])
                qv = plsc.load_gather(qt, [dv])
                Lt[pl.ds(base, LN)] = pv + qv
            pltpu.async_copy(Lt.at[pl.ds(ch * 128, 128)],
                             acc_sh.at[dst2.at[ch]], ssm, add=True)

        pltpu.sync_copy(Lt, L_hbm.at[pl.ds(wid * ET, ET)])

        @pl.loop(0, NCH)
        def _(ch):
            pltpu.make_async_copy(Lt.at[pl.ds(0, 128)],
                                  acc_sh.at[dst2.at[0]], ssm).wait()

        plsc.subcore_barrier()
        pltpu.sync_copy(acc_sh.at[pl.ds(sid * stripe, stripe)], zbuf)
        pltpu.sync_copy(zbuf,
                        s_hbm.at[pl.ds(c * NP + sid * stripe, stripe)])

    return k(p, q, srcp, dstp)


def _sc_final(L, srcp, dstp, s2, r0, r1, b3b):
    EP = srcp.shape[0]
    NN = r0.shape[0]
    NP = s2.shape[0] // NC
    ET = EP // NW

    @functools.partial(
        pl.kernel,
        out_type=jax.ShapeDtypeStruct((EP,), F32),
        mesh=_mesh(),
        compiler_params=_sc_params(),
        scratch_types=[
            pltpu.VMEM((NN,), F32),
            pltpu.VMEM((NN,), F32),
            pltpu.VMEM((NN,), F32),
            pltpu.VMEM((NN,), F32),
            pltpu.VMEM((ET,), I32),
            pltpu.VMEM((ET,), I32),
            pltpu.VMEM((ET,), F32),
            pltpu.VMEM((ET,), F32),
            pltpu.VMEM((LN,), F32),
            pltpu.SemaphoreType.DMA,
        ],
    )
    def k(L_hbm, src_hbm, dst_hbm, s_hbm, r0_hbm, r1_hbm, b3_hbm, out_hbm,
          st, tmp, r0t, r1t, src1, dst1, Lt, outt, b3t, sem):
        c = lax.axis_index("c")
        sid = lax.axis_index("s")
        wid = c * NS + sid

        copies = [
            (s_hbm.at[pl.ds(0, NN)], st),
            (s_hbm.at[pl.ds(NP, NN)], tmp),
            (r0_hbm, r0t),
            (r1_hbm, r1t),
            (src_hbm.at[pl.ds(wid * ET, ET)], src1),
            (dst_hbm.at[pl.ds(wid * ET, ET)], dst1),
            (L_hbm.at[pl.ds(wid * ET, ET)], Lt),
            (b3_hbm, b3t),
        ]
        for s_ref, d_ref in copies:
            pltpu.async_copy(s_ref, d_ref, sem)
        for s_ref, d_ref in copies:
            pltpu.make_async_copy(s_ref, d_ref, sem).wait()

        @pl.loop(0, NN, step=LN)
        def _(j):
            st[pl.ds(j, LN)] = st[pl.ds(j, LN)] + tmp[pl.ds(j, LN)]

        b3v = b3t[...]
        one = jnp.ones((LN,), F32)
        zero = jnp.zeros((LN,), F32)

        @pl.loop(0, ET, step=LN)
        def _(j):
            sv = src1[pl.ds(j, LN)]
            dv = dst1[pl.ds(j, LN)]
            Lv = Lt[pl.ds(j, LN)]
            selfm = sv == dv
            s_at = plsc.load_gather(st, [sv])
            r0v = plsc.load_gather(r0t, [sv])
            r1v = plsc.load_gather(r1t, [sv])
            rv = jnp.where(selfm, r1v, r0v)
            num = s_at - jnp.where(selfm, Lv, zero)
            z = Lv + num * rv + b3v
            outt[pl.ds(j, LN)] = 1.0 / (1.0 + jnp.exp(-z))

        pltpu.sync_copy(outt, out_hbm.at[pl.ds(wid * ET, ET)])

    return k(L, srcp, dstp, s2, r0, r1, b3b)


def kernel(x, edge_index, W1, b1, gamma1, beta1, W2, b2, W3, b3):
    N, D = x.shape
    E = edge_index.shape[1]
    NP = _round_up(N, 256)
    EP = _round_up(E, 128 * NW)

    src = edge_index[0]
    dst = edge_index[1]
    npad = EP - E
    padd = N + (jnp.arange(npad, dtype=I32) % (NP - N))
    pads = jnp.arange(npad, dtype=I32) % (NP - N)
    srcp = jnp.concatenate([src, pads])
    dstp = jnp.concatenate([dst, padd])

    w3a = W3[:D, 0]
    w3b = W3[D:, 0]
    b3b = jnp.broadcast_to(b3, (LN,)).astype(F32)

    asum1, cnt2 = _sc_rowscatter(x, srcp, dstp, NP, with_cnt=True)
    cnt = cnt2.reshape(NC, NP)
    h1, y2, degc = _tc_layer1(x, asum1, cnt, W1, b1, gamma1, beta1, W2, N)
    asum2, = _sc_rowscatter(h1, srcp, dstp, NP)
    p, q, r0, r1 = _tc_pq(y2, asum2, degc, b2, W2, w3a, w3b, N)
    L, s2 = _sc_edge_logits(p, q, srcp, dstp, NP)
    outp = _sc_final(L, srcp, dstp, s2, r0, r1, b3b)
    return outp[:E][:, None]


# speedup vs baseline: 18.3218x; 1.0014x over previous
import dataclasses
import functools

import jax
import jax.numpy as jnp
from jax import lax
from jax.experimental import pallas as pl
from jax.experimental.pallas import tpu as pltpu
from jax.experimental.pallas import tpu_sc as plsc

NC = 2
NS = 16
LN = 16
NW = NC * NS

F32 = jnp.float32
I32 = jnp.int32


def _round_up(v, m):
    return (v + m - 1) // m * m


def _mesh():
    return plsc.VectorSubcoreMesh(core_axis_name="c", subcore_axis_name="s")


def _sc_params():
    cp = pltpu.CompilerParams()
    if "needs_layout_passes" in pltpu.CompilerParams.__dataclass_fields__:
        cp = dataclasses.replace(cp, needs_layout_passes=False)
    return cp


def _tc_layer1(x, asum, cnt, W1, b1, g1, be1, W2, N):
    D = x.shape[1]

    def body(x_ref, a_ref, c_ref, w1_ref, b1_ref, g1_ref, be1_ref, w2_ref,
             h1_ref, y2_ref, deg_ref):
        deg = jnp.clip(c_ref[0, :N] + c_ref[1, :N], 1.0, None)
        asum_c = a_ref[0, :N] + a_ref[1, :N]
        y1 = jnp.dot(x_ref[...], w1_ref[...], preferred_element_type=F32)
        aggw = jnp.dot(asum_c, w1_ref[...],
                       preferred_element_type=F32) / deg[:, None]
        h = y1 + aggw + b1_ref[...][None, :]
        mu = jnp.mean(h, axis=0)
        var = jnp.mean((h - mu[None, :]) ** 2, axis=0)
        hn = (h - mu[None, :]) * lax.rsqrt(var + 1e-5)[None, :]
        hn = hn * g1_ref[...][None, :] + be1_ref[...][None, :]
        hr = jnp.maximum(hn, 0.0)
        h1_ref[...] = hr
        y2_ref[...] = jnp.dot(hr, w2_ref[...], preferred_element_type=F32)
        deg_ref[...] = deg

    return pl.pallas_call(
        body,
        out_shape=[jax.ShapeDtypeStruct((N, D), F32),
                   jax.ShapeDtypeStruct((N, D), F32),
                   jax.ShapeDtypeStruct((N,), F32)],
    )(x, asum, cnt, W1, b1, g1, be1, W2)


def _tc_pq(y2, asum, degc, b2, W2, w3a, w3b, N):
    D = y2.shape[1]

    def body(y_ref, a_ref, d_ref, b2_ref, w2_ref, wa_ref, wb_ref,
             p_ref, q_ref, r0_ref, r1_ref):
        deg = d_ref[...]
        asum_c = a_ref[0, :N] + a_ref[1, :N]
        aggw = jnp.dot(asum_c, w2_ref[...],
                       preferred_element_type=F32) / deg[:, None]
        h2 = jnp.maximum(y_ref[...] + aggw + b2_ref[...][None, :], 0.0)
        p_ref[...] = jnp.sum(h2 * wa_ref[...][None, :], axis=1)
        q_ref[...] = jnp.sum(h2 * wb_ref[...][None, :], axis=1)
        r0_ref[...] = 1.0 / deg
        r1_ref[...] = 1.0 / jnp.maximum(deg - 1.0, 1.0)

    return pl.pallas_call(
        body,
        out_shape=[jax.ShapeDtypeStruct((N,), F32),
                   jax.ShapeDtypeStruct((N,), F32),
                   jax.ShapeDtypeStruct((N,), F32),
                   jax.ShapeDtypeStruct((N,), F32)],
    )(y2, asum, degc, b2, W2, w3a, w3b)


def _sc_rowscatter(y, srcp, dstp, NP, with_cnt=False):
    D = y.shape[1]
    EP = srcp.shape[0]
    CH = 64
    NCH = EP // NW // CH
    stripe = NP // NS

    NB = 3
    G = 2
    out_type = [jax.ShapeDtypeStruct((NC, NP, D), F32)]
    scratch = [
        pltpu.VMEM((NCH, CH), I32),
        pltpu.VMEM((NCH, CH), I32),
    ]
    scratch += [pltpu.VMEM((CH, D), F32) for _ in range(NB)]
    scratch += [
        pltpu.VMEM_SHARED((NP, D), F32),
        pltpu.SemaphoreType.DMA,
    ]
    scratch += [pltpu.SemaphoreType.DMA for _ in range(NB)]
    scratch += [pltpu.SemaphoreType.DMA for _ in range(NB)]
    if with_cnt:
        out_type.append(jax.ShapeDtypeStruct((NC * NP,), F32))
        scratch += [
            pltpu.VMEM((CH,), F32),
            pltpu.VMEM((stripe,), F32),
            pltpu.VMEM_SHARED((NP,), F32),
            pltpu.SemaphoreType.DMA,
        ]

    @functools.partial(
        pl.kernel,
        out_type=out_type,
        mesh=_mesh(),
        compiler_params=_sc_params(),
        scratch_types=scratch,
    )
    def k(y_hbm, src_hbm, dst_hbm, out_hbm, *rest):
        if with_cnt:
            cnt_hbm = rest[0]
            rest = rest[1:]
        src2, dst2 = rest[0], rest[1]
        bufs = rest[2:2 + NB]
        acc_sh = rest[2 + NB]
        st = rest[3 + NB]
        gsem = rest[4 + NB:4 + 2 * NB]
        ssem = rest[4 + 2 * NB:4 + 3 * NB]
        if with_cnt:
            ones, zbuf, cnt_sh, csem = rest[4 + 3 * NB:]
        c = lax.axis_index("c")
        sid = lax.axis_index("s")
        wid = c * NS + sid
        rows0 = bufs[0]

        zero16 = jnp.zeros((LN,), F32)
        ones16 = jnp.ones((LN,), F32)

        @pl.loop(0, NCH)
        def _(t):
            base = wid * NCH * CH + t * CH
            pltpu.async_copy(src_hbm.at[pl.ds(base, CH)], src2.at[t], st)
            pltpu.async_copy(dst_hbm.at[pl.ds(base, CH)], dst2.at[t], st)

        @pl.loop(0, CH)
        def _(r):
            for kk in range(D // LN):
                rows0[r, pl.ds(kk * LN, LN)] = zero16

        for t in range(stripe // CH):
            pltpu.sync_copy(rows0,
                            acc_sh.at[pl.ds(sid * stripe + t * CH, CH)])

        if with_cnt:
            @pl.loop(0, CH, step=LN)
            def _(j):
                ones[pl.ds(j, LN)] = ones16

            @pl.loop(0, stripe, step=LN)
            def _(j):
                zbuf[pl.ds(j, LN)] = zero16

            pltpu.sync_copy(zbuf, cnt_sh.at[pl.ds(sid * stripe, stripe)])

        @pl.loop(0, NCH)
        def _(t):
            pltpu.make_async_copy(src_hbm.at[pl.ds(0, CH)], src2.at[t],
                                  st).wait()
            pltpu.make_async_copy(dst_hbm.at[pl.ds(0, CH)], dst2.at[t],
                                  st).wait()

        plsc.subcore_barrier()

        for g in range(G):
            pltpu.async_copy(y_hbm.at[src2.at[g]], bufs[g], gsem[g])
        for i in range(NCH):
            b = i % NB
            pltpu.make_async_copy(y_hbm.at[src2.at[i]], bufs[b],
                                  gsem[b]).wait()
            pltpu.async_copy(bufs[b], acc_sh.at[dst2.at[i]], ssem[b],
                             add=True)
            if with_cnt:
                pltpu.async_copy(ones, cnt_sh.at[dst2.at[i]], csem,
                                 add=True)
            if i + G < NCH:
                nb = (i + G) % NB
                if i + G - NB >= 0:
                    pltpu.make_async_copy(bufs[nb], acc_sh.at[dst2.at[0]],
                                          ssem[nb]).wait()
                pltpu.async_copy(y_hbm.at[src2.at[i + G]], bufs[nb],
                                 gsem[nb])
        for i in range(max(NCH - NB, 0), NCH):
            pltpu.make_async_copy(bufs[i % NB], acc_sh.at[dst2.at[0]],
                                  ssem[i % NB]).wait()
        if with_cnt:
            for i in range(NCH):
                pltpu.make_async_copy(ones, cnt_sh.at[dst2.at[0]],
                                      csem).wait()

        plsc.subcore_barrier()
        for t in range(stripe // CH):
            pltpu.sync_copy(acc_sh.at[pl.ds(sid * stripe + t * CH, CH)],
                            rows0)
            pltpu.sync_copy(rows0,
                            out_hbm.at[c, pl.ds(sid * stripe + t * CH, CH)])
        if with_cnt:
            pltpu.sync_copy(cnt_sh.at[pl.ds(sid * stripe, stripe)], zbuf)
            pltpu.sync_copy(zbuf,
                            cnt_hbm.at[pl.ds(c * NP + sid * stripe, stripe)])

    return k(y, srcp, dstp)


def _sc_edge_logits(p, q, srcp, dstp, NP):
    EP = srcp.shape[0]
    NN = p.shape[0]
    ET = EP // NW
    NCH = ET // 128
    stripe = NP // NS

    @functools.partial(
        pl.kernel,
        out_type=[jax.ShapeDtypeStruct((EP,), F32),
                  jax.ShapeDtypeStruct((NC * NP,), F32)],
        mesh=_mesh(),
        compiler_params=_sc_params(),
        scratch_types=[
            pltpu.VMEM((NN,), F32),
            pltpu.VMEM((NN,), F32),
            pltpu.VMEM((ET,), I32),
            pltpu.VMEM((ET,), I32),
            pltpu.VMEM((NCH, 128), I32),
            pltpu.VMEM((ET,), F32),
            pltpu.VMEM((stripe,), F32),
            pltpu.VMEM_SHARED((NP,), F32),
            pltpu.SemaphoreType.DMA,
            pltpu.SemaphoreType.DMA,
        ],
    )
    def k(p_hbm, q_hbm, src_hbm, dst_hbm, L_hbm, s_hbm,
          pt, qt, src1, dst1, dst2, Lt, zbuf, acc_sh, st, ssm):
        c = lax.axis_index("c")
        sid = lax.axis_index("s")
        wid = c * NS + sid

        zero16 = jnp.zeros((LN,), F32)

        pltpu.async_copy(p_hbm, pt, st)
        pltpu.async_copy(q_hbm, qt, st)
        pltpu.async_copy(src_hbm.at[pl.ds(wid * ET, ET)], src1, st)
        pltpu.async_copy(dst_hbm.at[pl.ds(wid * ET, ET)], dst1, st)

        @pl.loop(0, NCH)
        def _(t):
            pltpu.async_copy(dst_hbm.at[pl.ds(wid * ET + t * 128, 128)],
                             dst2.at[t], st)

        @pl.loop(0, stripe, step=LN)
        def _(j):
            zbuf[pl.ds(j, LN)] = zero16

        pltpu.sync_copy(zbuf, acc_sh.at[pl.ds(sid * stripe, stripe)])
        pltpu.make_async_copy(p_hbm, pt, st).wait()
        pltpu.make_async_copy(q_hbm, qt, st).wait()
        pltpu.make_async_copy(src_hbm.at[pl.ds(wid * ET, ET)], src1,
                              st).wait()
        pltpu.make_async_copy(dst_hbm.at[pl.ds(wid * ET, ET)], dst1,
                              st).wait()

        @pl.loop(0, NCH)
        def _(t):
            pltpu.make_async_copy(dst_hbm.at[pl.ds(0, 128)], dst2.at[t],
                                  st).wait()

        plsc.subcore_barrier()

        @pl.loop(0, NCH)
        def _(ch):
            for g in range(8):
                base = ch * 128 + g * 16
                sv = src1[pl.ds(base, LN)]
                dv = dst1[pl.ds(base, LN)]
                pv = plsc.load_gather(pt, [sv])
                qv = plsc.load_gather(qt, [dv])
                Lt[pl.ds(base, LN)] = pv + qv
            pltpu.async_copy(Lt.at[pl.ds(ch * 128, 128)],
                             acc_sh.at[dst2.at[ch]], ssm, add=True)

        pltpu.sync_copy(Lt, L_hbm.at[pl.ds(wid * ET, ET)])

        @pl.loop(0, NCH)
        def _(ch):
            pltpu.make_async_copy(Lt.at[pl.ds(0, 128)],
                                  acc_sh.at[dst2.at[0]], ssm).wait()

        plsc.subcore_barrier()
        pltpu.sync_copy(acc_sh.at[pl.ds(sid * stripe, stripe)], zbuf)
        pltpu.sync_copy(zbuf,
                        s_hbm.at[pl.ds(c * NP + sid * stripe, stripe)])

    return k(p, q, srcp, dstp)


def _sc_final(L, srcp, dstp, s2, r0, r1, b3b):
    EP = srcp.shape[0]
    NN = r0.shape[0]
    NP = s2.shape[0] // NC
    ET = EP // NW

    @functools.partial(
        pl.kernel,
        out_type=jax.ShapeDtypeStruct((EP,), F32),
        mesh=_mesh(),
        compiler_params=_sc_params(),
        scratch_types=[
            pltpu.VMEM((NN,), F32),
            pltpu.VMEM((NN,), F32),
            pltpu.VMEM((NN,), F32),
            pltpu.VMEM((NN,), F32),
            pltpu.VMEM((ET,), I32),
            pltpu.VMEM((ET,), I32),
            pltpu.VMEM((ET,), F32),
            pltpu.VMEM((ET,), F32),
            pltpu.VMEM((LN,), F32),
            pltpu.SemaphoreType.DMA,
        ],
    )
    def k(L_hbm, src_hbm, dst_hbm, s_hbm, r0_hbm, r1_hbm, b3_hbm, out_hbm,
          st, tmp, r0t, r1t, src1, dst1, Lt, outt, b3t, sem):
        c = lax.axis_index("c")
        sid = lax.axis_index("s")
        wid = c * NS + sid

        copies = [
            (s_hbm.at[pl.ds(0, NN)], st),
            (s_hbm.at[pl.ds(NP, NN)], tmp),
            (r0_hbm, r0t),
            (r1_hbm, r1t),
            (src_hbm.at[pl.ds(wid * ET, ET)], src1),
            (dst_hbm.at[pl.ds(wid * ET, ET)], dst1),
            (L_hbm.at[pl.ds(wid * ET, ET)], Lt),
            (b3_hbm, b3t),
        ]
        for s_ref, d_ref in copies:
            pltpu.async_copy(s_ref, d_ref, sem)
        for s_ref, d_ref in copies:
            pltpu.make_async_copy(s_ref, d_ref, sem).wait()

        @pl.loop(0, NN, step=LN)
        def _(j):
            st[pl.ds(j, LN)] = st[pl.ds(j, LN)] + tmp[pl.ds(j, LN)]

        b3v = b3t[...]
        one = jnp.ones((LN,), F32)
        zero = jnp.zeros((LN,), F32)

        @pl.loop(0, ET, step=2 * LN)
        def _(j):
            for u in range(2):
                jj = j + u * LN
                sv = src1[pl.ds(jj, LN)]
                dv = dst1[pl.ds(jj, LN)]
                Lv = Lt[pl.ds(jj, LN)]
                selfm = sv == dv
                s_at = plsc.load_gather(st, [sv])
                r0v = plsc.load_gather(r0t, [sv])
                r1v = plsc.load_gather(r1t, [sv])
                rv = jnp.where(selfm, r1v, r0v)
                num = s_at - jnp.where(selfm, Lv, zero)
                z = Lv + num * rv + b3v
                outt[pl.ds(jj, LN)] = 1.0 / (1.0 + jnp.exp(-z))

        pltpu.sync_copy(outt, out_hbm.at[pl.ds(wid * ET, ET)])

    return k(L, srcp, dstp, s2, r0, r1, b3b)


def kernel(x, edge_index, W1, b1, gamma1, beta1, W2, b2, W3, b3):
    N, D = x.shape
    E = edge_index.shape[1]
    NP = _round_up(N, 256)
    EP = _round_up(E, 128 * NW)

    src = edge_index[0]
    dst = edge_index[1]
    npad = EP - E
    padd = N + (jnp.arange(npad, dtype=I32) % (NP - N))
    pads = jnp.arange(npad, dtype=I32) % (NP - N)
    srcp = jnp.concatenate([src, pads])
    dstp = jnp.concatenate([dst, padd])

    w3a = W3[:D, 0]
    w3b = W3[D:, 0]
    b3b = jnp.broadcast_to(b3, (LN,)).astype(F32)

    asum1, cnt2 = _sc_rowscatter(x, srcp, dstp, NP, with_cnt=True)
    cnt = cnt2.reshape(NC, NP)
    h1, y2, degc = _tc_layer1(x, asum1, cnt, W1, b1, gamma1, beta1, W2, N)
    asum2, = _sc_rowscatter(h1, srcp, dstp, NP)
    p, q, r0, r1 = _tc_pq(y2, asum2, degc, b2, W2, w3a, w3b, N)
    L, s2 = _sc_edge_logits(p, q, srcp, dstp, NP)
    outp = _sc_final(L, srcp, dstp, s2, r0, r1, b3b)
    return outp[:E][:, None]


# speedup vs baseline: 18.5719x; 1.0137x over previous
import dataclasses
import functools

import jax
import jax.numpy as jnp
from jax import lax
from jax.experimental import pallas as pl
from jax.experimental.pallas import tpu as pltpu
from jax.experimental.pallas import tpu_sc as plsc

NC = 2
NS = 16
LN = 16
NW = NC * NS

F32 = jnp.float32
I32 = jnp.int32


def _round_up(v, m):
    return (v + m - 1) // m * m


def _mesh():
    return plsc.VectorSubcoreMesh(core_axis_name="c", subcore_axis_name="s")


def _sc_params():
    cp = pltpu.CompilerParams()
    if "needs_layout_passes" in pltpu.CompilerParams.__dataclass_fields__:
        cp = dataclasses.replace(cp, needs_layout_passes=False)
    return cp


def _tc_layer1(x, asum, cnt, W1, b1, g1, be1, W2, N):
    D = x.shape[1]

    def body(x_ref, a_ref, c_ref, w1_ref, b1_ref, g1_ref, be1_ref, w2_ref,
             h1_ref, y2_ref, deg_ref):
        deg = jnp.clip(c_ref[0, :N] + c_ref[1, :N], 1.0, None)
        asum_c = a_ref[0, :N] + a_ref[1, :N]
        y1 = jnp.dot(x_ref[...], w1_ref[...], preferred_element_type=F32)
        aggw = jnp.dot(asum_c, w1_ref[...],
                       preferred_element_type=F32) / deg[:, None]
        h = y1 + aggw + b1_ref[...][None, :]
        mu = jnp.mean(h, axis=0)
        var = jnp.mean((h - mu[None, :]) ** 2, axis=0)
        hn = (h - mu[None, :]) * lax.rsqrt(var + 1e-5)[None, :]
        hn = hn * g1_ref[...][None, :] + be1_ref[...][None, :]
        hr = jnp.maximum(hn, 0.0)
        h1_ref[...] = hr
        y2_ref[...] = jnp.dot(hr, w2_ref[...], preferred_element_type=F32)
        deg_ref[...] = deg

    return pl.pallas_call(
        body,
        out_shape=[jax.ShapeDtypeStruct((N, D), F32),
                   jax.ShapeDtypeStruct((N, D), F32),
                   jax.ShapeDtypeStruct((N,), F32)],
    )(x, asum, cnt, W1, b1, g1, be1, W2)


def _tc_pq(y2, asum, degc, b2, W2, w3a, w3b, N):
    D = y2.shape[1]

    def body(y_ref, a_ref, d_ref, b2_ref, w2_ref, wa_ref, wb_ref,
             p_ref, q_ref):
        deg = d_ref[...]
        asum_c = a_ref[0, :N] + a_ref[1, :N]
        aggw = jnp.dot(asum_c, w2_ref[...],
                       preferred_element_type=F32) / deg[:, None]
        h2 = jnp.maximum(y_ref[...] + aggw + b2_ref[...][None, :], 0.0)
        p_ref[...] = jnp.sum(h2 * wa_ref[...][None, :], axis=1)
        q_ref[...] = jnp.sum(h2 * wb_ref[...][None, :], axis=1)

    return pl.pallas_call(
        body,
        out_shape=[jax.ShapeDtypeStruct((N,), F32),
                   jax.ShapeDtypeStruct((N,), F32)],
    )(y2, asum, degc, b2, W2, w3a, w3b)


def _sc_rowscatter(y, srcp, dstp, NP, with_cnt=False):
    D = y.shape[1]
    EP = srcp.shape[0]
    CH = 64
    NCH = EP // NW // CH
    stripe = NP // NS

    NB = 3
    G = 2
    out_type = [jax.ShapeDtypeStruct((NC, NP, D), F32)]
    scratch = [
        pltpu.VMEM((NCH, CH), I32),
        pltpu.VMEM((NCH, CH), I32),
    ]
    scratch += [pltpu.VMEM((CH, D), F32) for _ in range(NB)]
    scratch += [
        pltpu.VMEM_SHARED((NP, D), F32),
        pltpu.SemaphoreType.DMA,
    ]
    scratch += [pltpu.SemaphoreType.DMA for _ in range(NB)]
    scratch += [pltpu.SemaphoreType.DMA for _ in range(NB)]
    if with_cnt:
        out_type.append(jax.ShapeDtypeStruct((NC * NP,), F32))
        scratch += [
            pltpu.VMEM((CH,), F32),
            pltpu.VMEM((stripe,), F32),
            pltpu.VMEM_SHARED((NP,), F32),
            pltpu.SemaphoreType.DMA,
        ]

    @functools.partial(
        pl.kernel,
        out_type=out_type,
        mesh=_mesh(),
        compiler_params=_sc_params(),
        scratch_types=scratch,
    )
    def k(y_hbm, src_hbm, dst_hbm, out_hbm, *rest):
        if with_cnt:
            cnt_hbm = rest[0]
            rest = rest[1:]
        src2, dst2 = rest[0], rest[1]
        bufs = rest[2:2 + NB]
        acc_sh = rest[2 + NB]
        st = rest[3 + NB]
        gsem = rest[4 + NB:4 + 2 * NB]
        ssem = rest[4 + 2 * NB:4 + 3 * NB]
        if with_cnt:
            ones, zbuf, cnt_sh, csem = rest[4 + 3 * NB:]
        c = lax.axis_index("c")
        sid = lax.axis_index("s")
        wid = c * NS + sid
        rows0 = bufs[0]

        zero16 = jnp.zeros((LN,), F32)
        ones16 = jnp.ones((LN,), F32)

        @pl.loop(0, NCH)
        def _(t):
            base = wid * NCH * CH + t * CH
            pltpu.async_copy(src_hbm.at[pl.ds(base, CH)], src2.at[t], st)
            pltpu.async_copy(dst_hbm.at[pl.ds(base, CH)], dst2.at[t], st)

        @pl.loop(0, CH)
        def _(r):
            for kk in range(D // LN):
                rows0[r, pl.ds(kk * LN, LN)] = zero16

        for t in range(stripe // CH):
            pltpu.sync_copy(rows0,
                            acc_sh.at[pl.ds(sid * stripe + t * CH, CH)])

        if with_cnt:
            @pl.loop(0, CH, step=LN)
            def _(j):
                ones[pl.ds(j, LN)] = ones16

            @pl.loop(0, stripe, step=LN)
            def _(j):
                zbuf[pl.ds(j, LN)] = zero16

            pltpu.sync_copy(zbuf, cnt_sh.at[pl.ds(sid * stripe, stripe)])

        @pl.loop(0, NCH)
        def _(t):
            pltpu.make_async_copy(src_hbm.at[pl.ds(0, CH)], src2.at[t],
                                  st).wait()
            pltpu.make_async_copy(dst_hbm.at[pl.ds(0, CH)], dst2.at[t],
                                  st).wait()

        plsc.subcore_barrier()

        for g in range(G):
            pltpu.async_copy(y_hbm.at[src2.at[g]], bufs[g], gsem[g])
        for i in range(NCH):
            b = i % NB
            pltpu.make_async_copy(y_hbm.at[src2.at[i]], bufs[b],
                                  gsem[b]).wait()
            pltpu.async_copy(bufs[b], acc_sh.at[dst2.at[i]], ssem[b],
                             add=True)
            if with_cnt:
                pltpu.async_copy(ones, cnt_sh.at[dst2.at[i]], csem,
                                 add=True)
            if i + G < NCH:
                nb = (i + G) % NB
                if i + G - NB >= 0:
                    pltpu.make_async_copy(bufs[nb], acc_sh.at[dst2.at[0]],
                                          ssem[nb]).wait()
                pltpu.async_copy(y_hbm.at[src2.at[i + G]], bufs[nb],
                                 gsem[nb])
        for i in range(max(NCH - NB, 0), NCH):
            pltpu.make_async_copy(bufs[i % NB], acc_sh.at[dst2.at[0]],
                                  ssem[i % NB]).wait()
        if with_cnt:
            for i in range(NCH):
                pltpu.make_async_copy(ones, cnt_sh.at[dst2.at[0]],
                                      csem).wait()

        plsc.subcore_barrier()
        for t in range(stripe // CH):
            pltpu.sync_copy(acc_sh.at[pl.ds(sid * stripe + t * CH, CH)],
                            rows0)
            pltpu.sync_copy(rows0,
                            out_hbm.at[c, pl.ds(sid * stripe + t * CH, CH)])
        if with_cnt:
            pltpu.sync_copy(cnt_sh.at[pl.ds(sid * stripe, stripe)], zbuf)
            pltpu.sync_copy(zbuf,
                            cnt_hbm.at[pl.ds(c * NP + sid * stripe, stripe)])

    return k(y, srcp, dstp)


def _sc_edge_logits(p, q, srcp, dstp, NP):
    EP = srcp.shape[0]
    NN = p.shape[0]
    ET = EP // NW
    NCH = ET // 128
    stripe = NP // NS

    @functools.partial(
        pl.kernel,
        out_type=[jax.ShapeDtypeStruct((EP,), F32),
                  jax.ShapeDtypeStruct((NC * NP,), F32)],
        mesh=_mesh(),
        compiler_params=_sc_params(),
        scratch_types=[
            pltpu.VMEM((NN,), F32),
            pltpu.VMEM((NN,), F32),
            pltpu.VMEM((ET,), I32),
            pltpu.VMEM((ET,), I32),
            pltpu.VMEM((NCH, 128), I32),
            pltpu.VMEM((ET,), F32),
            pltpu.VMEM((stripe,), F32),
            pltpu.VMEM_SHARED((NP,), F32),
            pltpu.SemaphoreType.DMA,
            pltpu.SemaphoreType.DMA,
        ],
    )
    def k(p_hbm, q_hbm, src_hbm, dst_hbm, L_hbm, s_hbm,
          pt, qt, src1, dst1, dst2, Lt, zbuf, acc_sh, st, ssm):
        c = lax.axis_index("c")
        sid = lax.axis_index("s")
        wid = c * NS + sid

        zero16 = jnp.zeros((LN,), F32)

        pltpu.async_copy(p_hbm, pt, st)
        pltpu.async_copy(q_hbm, qt, st)
        pltpu.async_copy(src_hbm.at[pl.ds(wid * ET, ET)], src1, st)
        pltpu.async_copy(dst_hbm.at[pl.ds(wid * ET, ET)], dst1, st)

        @pl.loop(0, NCH)
        def _(t):
            pltpu.async_copy(dst_hbm.at[pl.ds(wid * ET + t * 128, 128)],
                             dst2.at[t], st)

        @pl.loop(0, stripe, step=LN)
        def _(j):
            zbuf[pl.ds(j, LN)] = zero16

        pltpu.sync_copy(zbuf, acc_sh.at[pl.ds(sid * stripe, stripe)])
        pltpu.make_async_copy(p_hbm, pt, st).wait()
        pltpu.make_async_copy(q_hbm, qt, st).wait()
        pltpu.make_async_copy(src_hbm.at[pl.ds(wid * ET, ET)], src1,
                              st).wait()
        pltpu.make_async_copy(dst_hbm.at[pl.ds(wid * ET, ET)], dst1,
                              st).wait()

        @pl.loop(0, NCH)
        def _(t):
            pltpu.make_async_copy(dst_hbm.at[pl.ds(0, 128)], dst2.at[t],
                                  st).wait()

        plsc.subcore_barrier()

        @pl.loop(0, NCH)
        def _(ch):
            for g in range(8):
                base = ch * 128 + g * 16
                sv = src1[pl.ds(base, LN)]
                dv = dst1[pl.ds(base, LN)]
                pv = plsc.load_gather(pt, [sv])
                qv = plsc.load_gather(qt, [dv])
                Lt[pl.ds(base, LN)] = pv + qv
            pltpu.async_copy(Lt.at[pl.ds(ch * 128, 128)],
                             acc_sh.at[dst2.at[ch]], ssm, add=True)

        pltpu.sync_copy(Lt, L_hbm.at[pl.ds(wid * ET, ET)])

        @pl.loop(0, NCH)
        def _(ch):
            pltpu.make_async_copy(Lt.at[pl.ds(0, 128)],
                                  acc_sh.at[dst2.at[0]], ssm).wait()

        plsc.subcore_barrier()
        pltpu.sync_copy(acc_sh.at[pl.ds(sid * stripe, stripe)], zbuf)
        pltpu.sync_copy(zbuf,
                        s_hbm.at[pl.ds(c * NP + sid * stripe, stripe)])

    return k(p, q, srcp, dstp)


def _sc_final(L, srcp, dstp, s2, degc, b3b):
    EP = srcp.shape[0]
    NN = degc.shape[0]
    NP = s2.shape[0] // NC
    ET = EP // NW

    @functools.partial(
        pl.kernel,
        out_type=jax.ShapeDtypeStruct((EP,), F32),
        mesh=_mesh(),
        compiler_params=_sc_params(),
        scratch_types=[
            pltpu.VMEM((NN,), F32),
            pltpu.VMEM((NN,), F32),
            pltpu.VMEM((NN,), F32),
            pltpu.VMEM((ET,), I32),
            pltpu.VMEM((ET,), I32),
            pltpu.VMEM((ET,), F32),
            pltpu.VMEM((ET,), F32),
            pltpu.VMEM((LN,), F32),
            pltpu.SemaphoreType.DMA,
        ],
    )
    def k(L_hbm, src_hbm, dst_hbm, s_hbm, deg_hbm, b3_hbm, out_hbm,
          st, tmp, degt, src1, dst1, Lt, outt, b3t, sem):
        c = lax.axis_index("c")
        sid = lax.axis_index("s")
        wid = c * NS + sid

        copies = [
            (s_hbm.at[pl.ds(0, NN)], st),
            (s_hbm.at[pl.ds(NP, NN)], tmp),
            (deg_hbm, degt),
            (src_hbm.at[pl.ds(wid * ET, ET)], src1),
            (dst_hbm.at[pl.ds(wid * ET, ET)], dst1),
            (L_hbm.at[pl.ds(wid * ET, ET)], Lt),
            (b3_hbm, b3t),
        ]
        for s_ref, d_ref in copies:
            pltpu.async_copy(s_ref, d_ref, sem)
        for s_ref, d_ref in copies:
            pltpu.make_async_copy(s_ref, d_ref, sem).wait()

        b3v = b3t[...]
        one = jnp.ones((LN,), F32)
        zero = jnp.zeros((LN,), F32)

        @pl.loop(0, ET, step=2 * LN)
        def _(j):
            for u in range(2):
                jj = j + u * LN
                sv = src1[pl.ds(jj, LN)]
                dv = dst1[pl.ds(jj, LN)]
                Lv = Lt[pl.ds(jj, LN)]
                selfm = sv == dv
                s_at = (plsc.load_gather(st, [sv])
                        + plsc.load_gather(tmp, [sv]))
                d_at = plsc.load_gather(degt, [sv])
                degv = jnp.maximum(d_at - jnp.where(selfm, one, zero), 1.0)
                num = s_at - jnp.where(selfm, Lv, zero)
                z = Lv + num / degv + b3v
                outt[pl.ds(jj, LN)] = 1.0 / (1.0 + jnp.exp(-z))

        pltpu.sync_copy(outt, out_hbm.at[pl.ds(wid * ET, ET)])

    return k(L, srcp, dstp, s2, degc, b3b)


def kernel(x, edge_index, W1, b1, gamma1, beta1, W2, b2, W3, b3):
    N, D = x.shape
    E = edge_index.shape[1]
    NP = _round_up(N, 256)
    EP = _round_up(E, 128 * NW)

    src = edge_index[0]
    dst = edge_index[1]
    npad = EP - E
    padd = N + (jnp.arange(npad, dtype=I32) % (NP - N))
    pads = jnp.arange(npad, dtype=I32) % (NP - N)
    srcp = jnp.concatenate([src, pads])
    dstp = jnp.concatenate([dst, padd])

    w3a = W3[:D, 0]
    w3b = W3[D:, 0]
    b3b = jnp.broadcast_to(b3, (LN,)).astype(F32)

    asum1, cnt2 = _sc_rowscatter(x, srcp, dstp, NP, with_cnt=True)
    cnt = cnt2.reshape(NC, NP)
    h1, y2, degc = _tc_layer1(x, asum1, cnt, W1, b1, gamma1, beta1, W2, N)
    asum2, = _sc_rowscatter(h1, srcp, dstp, NP)
    p, q = _tc_pq(y2, asum2, degc, b2, W2, w3a, w3b, N)
    L, s2 = _sc_edge_logits(p, q, srcp, dstp, NP)
    outp = _sc_final(L, srcp, dstp, s2, degc, b3b)
    return outp[:E][:, None]


# speedup vs baseline: 19.2076x; 1.0342x over previous
import dataclasses
import functools

import jax
import jax.numpy as jnp
from jax import lax
from jax.experimental import pallas as pl
from jax.experimental.pallas import tpu as pltpu
from jax.experimental.pallas import tpu_sc as plsc

NC = 2
NS = 16
LN = 16
NW = NC * NS

F32 = jnp.float32
I32 = jnp.int32


def _round_up(v, m):
    return (v + m - 1) // m * m


def _mesh():
    return plsc.VectorSubcoreMesh(core_axis_name="c", subcore_axis_name="s")


def _sc_params():
    cp = pltpu.CompilerParams()
    if "needs_layout_passes" in pltpu.CompilerParams.__dataclass_fields__:
        cp = dataclasses.replace(cp, needs_layout_passes=False)
    return cp


def _tc_layer1(x, asum, cnt, W1, b1, g1, be1, W2, N):
    D = x.shape[1]

    def body(x_ref, a_ref, c_ref, w1_ref, b1_ref, g1_ref, be1_ref, w2_ref,
             h1_ref, y2_ref, deg_ref):
        deg = jnp.clip(c_ref[0, :N] + c_ref[1, :N], 1.0, None)
        asum_c = a_ref[0, :N] + a_ref[1, :N]
        y1 = jnp.dot(x_ref[...], w1_ref[...], preferred_element_type=F32)
        aggw = jnp.dot(asum_c, w1_ref[...],
                       preferred_element_type=F32) / deg[:, None]
        h = y1 + aggw + b1_ref[...][None, :]
        mu = jnp.mean(h, axis=0)
        var = jnp.mean((h - mu[None, :]) ** 2, axis=0)
        hn = (h - mu[None, :]) * lax.rsqrt(var + 1e-5)[None, :]
        hn = hn * g1_ref[...][None, :] + be1_ref[...][None, :]
        hr = jnp.maximum(hn, 0.0)
        h1_ref[...] = hr
        y2_ref[...] = jnp.dot(hr, w2_ref[...], preferred_element_type=F32)
        deg_ref[...] = deg

    return pl.pallas_call(
        body,
        out_shape=[jax.ShapeDtypeStruct((N, D), F32),
                   jax.ShapeDtypeStruct((N, D), F32),
                   jax.ShapeDtypeStruct((N,), F32)],
    )(x, asum, cnt, W1, b1, g1, be1, W2)


def _tc_pq(y2, asum, degc, b2, W2, w3a, w3b, N):
    D = y2.shape[1]

    def body(y_ref, a_ref, d_ref, b2_ref, w2_ref, wa_ref, wb_ref,
             p_ref, q_ref):
        deg = d_ref[...]
        asum_c = a_ref[0, :N] + a_ref[1, :N]
        aggw = jnp.dot(asum_c, w2_ref[...],
                       preferred_element_type=F32) / deg[:, None]
        h2 = jnp.maximum(y_ref[...] + aggw + b2_ref[...][None, :], 0.0)
        p_ref[...] = jnp.sum(h2 * wa_ref[...][None, :], axis=1)
        q_ref[...] = jnp.sum(h2 * wb_ref[...][None, :], axis=1)

    return pl.pallas_call(
        body,
        out_shape=[jax.ShapeDtypeStruct((N,), F32),
                   jax.ShapeDtypeStruct((N,), F32)],
    )(y2, asum, degc, b2, W2, w3a, w3b)


def _sc_rowscatter(y, srcp, dstp, NP, with_cnt=False):
    D = y.shape[1]
    EP = srcp.shape[0]
    CH = 64
    NCH = EP // NW // CH
    stripe = NP // NS

    NB = 3
    G = 2
    out_type = [jax.ShapeDtypeStruct((NC, NP, D), F32)]
    scratch = [
        pltpu.VMEM((NCH, CH), I32),
        pltpu.VMEM((NCH, CH), I32),
    ]
    scratch += [pltpu.VMEM((CH, D), F32) for _ in range(NB)]
    scratch += [
        pltpu.VMEM_SHARED((NP, D), F32),
        pltpu.SemaphoreType.DMA,
    ]
    scratch += [pltpu.SemaphoreType.DMA for _ in range(NB)]
    scratch += [pltpu.SemaphoreType.DMA for _ in range(NB)]
    if with_cnt:
        out_type.append(jax.ShapeDtypeStruct((NC * NP,), F32))
        scratch += [
            pltpu.VMEM((CH,), F32),
            pltpu.VMEM((stripe,), F32),
            pltpu.VMEM_SHARED((NP,), F32),
            pltpu.SemaphoreType.DMA,
        ]

    @functools.partial(
        pl.kernel,
        out_type=out_type,
        mesh=_mesh(),
        compiler_params=_sc_params(),
        scratch_types=scratch,
    )
    def k(y_hbm, src_hbm, dst_hbm, out_hbm, *rest):
        if with_cnt:
            cnt_hbm = rest[0]
            rest = rest[1:]
        src2, dst2 = rest[0], rest[1]
        bufs = rest[2:2 + NB]
        acc_sh = rest[2 + NB]
        st = rest[3 + NB]
        gsem = rest[4 + NB:4 + 2 * NB]
        ssem = rest[4 + 2 * NB:4 + 3 * NB]
        if with_cnt:
            ones, zbuf, cnt_sh, csem = rest[4 + 3 * NB:]
        c = lax.axis_index("c")
        sid = lax.axis_index("s")
        wid = c * NS + sid
        rows0 = bufs[0]

        zero16 = jnp.zeros((LN,), F32)
        ones16 = jnp.ones((LN,), F32)

        @pl.loop(0, NCH)
        def _(t):
            base = wid * NCH * CH + t * CH
            pltpu.async_copy(src_hbm.at[pl.ds(base, CH)], src2.at[t], st)
            pltpu.async_copy(dst_hbm.at[pl.ds(base, CH)], dst2.at[t], st)

        @pl.loop(0, CH)
        def _(r):
            for kk in range(D // LN):
                rows0[r, pl.ds(kk * LN, LN)] = zero16

        for t in range(stripe // CH):
            pltpu.async_copy(rows0,
                             acc_sh.at[pl.ds(sid * stripe + t * CH, CH)], st)
        for t in range(stripe // CH):
            pltpu.make_async_copy(rows0,
                                  acc_sh.at[pl.ds(sid * stripe, CH)],
                                  st).wait()

        if with_cnt:
            @pl.loop(0, CH, step=LN)
            def _(j):
                ones[pl.ds(j, LN)] = ones16

            @pl.loop(0, stripe, step=LN)
            def _(j):
                zbuf[pl.ds(j, LN)] = zero16

            pltpu.sync_copy(zbuf, cnt_sh.at[pl.ds(sid * stripe, stripe)])

        @pl.loop(0, NCH)
        def _(t):
            pltpu.make_async_copy(src_hbm.at[pl.ds(0, CH)], src2.at[t],
                                  st).wait()
            pltpu.make_async_copy(dst_hbm.at[pl.ds(0, CH)], dst2.at[t],
                                  st).wait()

        plsc.subcore_barrier()

        for g in range(G):
            pltpu.async_copy(y_hbm.at[src2.at[g]], bufs[g], gsem[g])
        for i in range(NCH):
            b = i % NB
            pltpu.make_async_copy(y_hbm.at[src2.at[i]], bufs[b],
                                  gsem[b]).wait()
            pltpu.async_copy(bufs[b], acc_sh.at[dst2.at[i]], ssem[b],
                             add=True)
            if with_cnt:
                pltpu.async_copy(ones, cnt_sh.at[dst2.at[i]], csem,
                                 add=True)
            if i + G < NCH:
                nb = (i + G) % NB
                if i + G - NB >= 0:
                    pltpu.make_async_copy(bufs[nb], acc_sh.at[dst2.at[0]],
                                          ssem[nb]).wait()
                pltpu.async_copy(y_hbm.at[src2.at[i + G]], bufs[nb],
                                 gsem[nb])
        for i in range(max(NCH - NB, 0), NCH):
            pltpu.make_async_copy(bufs[i % NB], acc_sh.at[dst2.at[0]],
                                  ssem[i % NB]).wait()
        if with_cnt:
            for i in range(NCH):
                pltpu.make_async_copy(ones, cnt_sh.at[dst2.at[0]],
                                      csem).wait()

        plsc.subcore_barrier()
        NT = stripe // CH
        if with_cnt:
            pltpu.async_copy(cnt_sh.at[pl.ds(sid * stripe, stripe)], zbuf,
                             csem)
        pltpu.async_copy(acc_sh.at[pl.ds(sid * stripe, CH)], bufs[0],
                         gsem[0])
        for t in range(NT):
            b = t % 2
            pltpu.make_async_copy(acc_sh.at[pl.ds(sid * stripe, CH)],
                                  bufs[b], gsem[b]).wait()
            pltpu.async_copy(bufs[b],
                             out_hbm.at[c, pl.ds(sid * stripe + t * CH, CH)],
                             ssem[b])
            if t + 1 < NT:
                nb = (t + 1) % 2
                if t - 1 >= 0:
                    pltpu.make_async_copy(
                        bufs[nb],
                        out_hbm.at[c, pl.ds(sid * stripe, CH)],
                        ssem[nb]).wait()
                pltpu.async_copy(
                    acc_sh.at[pl.ds(sid * stripe + (t + 1) * CH, CH)],
                    bufs[nb], gsem[nb])
        for t in range(NT - 2, NT):
            pltpu.make_async_copy(bufs[t % 2],
                                  out_hbm.at[c, pl.ds(sid * stripe, CH)],
                                  ssem[t % 2]).wait()
        if with_cnt:
            pltpu.make_async_copy(cnt_sh.at[pl.ds(sid * stripe, stripe)],
                                  zbuf, csem).wait()
            pltpu.sync_copy(zbuf,
                            cnt_hbm.at[pl.ds(c * NP + sid * stripe, stripe)])

    return k(y, srcp, dstp)


def _sc_edge_logits(p, q, srcp, dstp, NP):
    EP = srcp.shape[0]
    NN = p.shape[0]
    ET = EP // NW
    NCH = ET // 128
    stripe = NP // NS

    @functools.partial(
        pl.kernel,
        out_type=[jax.ShapeDtypeStruct((EP,), F32),
                  jax.ShapeDtypeStruct((NC * NP,), F32)],
        mesh=_mesh(),
        compiler_params=_sc_params(),
        scratch_types=[
            pltpu.VMEM((NN,), F32),
            pltpu.VMEM((NN,), F32),
            pltpu.VMEM((ET,), I32),
            pltpu.VMEM((ET,), I32),
            pltpu.VMEM((NCH, 128), I32),
            pltpu.VMEM((ET,), F32),
            pltpu.VMEM((stripe,), F32),
            pltpu.VMEM_SHARED((NP,), F32),
            pltpu.SemaphoreType.DMA,
            pltpu.SemaphoreType.DMA,
        ],
    )
    def k(p_hbm, q_hbm, src_hbm, dst_hbm, L_hbm, s_hbm,
          pt, qt, src1, dst1, dst2, Lt, zbuf, acc_sh, st, ssm):
        c = lax.axis_index("c")
        sid = lax.axis_index("s")
        wid = c * NS + sid

        zero16 = jnp.zeros((LN,), F32)

        pltpu.async_copy(p_hbm, pt, st)
        pltpu.async_copy(q_hbm, qt, st)
        pltpu.async_copy(src_hbm.at[pl.ds(wid * ET, ET)], src1, st)
        pltpu.async_copy(dst_hbm.at[pl.ds(wid * ET, ET)], dst1, st)

        @pl.loop(0, NCH)
        def _(t):
            pltpu.async_copy(dst_hbm.at[pl.ds(wid * ET + t * 128, 128)],
                             dst2.at[t], st)

        @pl.loop(0, stripe, step=LN)
        def _(j):
            zbuf[pl.ds(j, LN)] = zero16

        pltpu.sync_copy(zbuf, acc_sh.at[pl.ds(sid * stripe, stripe)])
        pltpu.make_async_copy(p_hbm, pt, st).wait()
        pltpu.make_async_copy(q_hbm, qt, st).wait()
        pltpu.make_async_copy(src_hbm.at[pl.ds(wid * ET, ET)], src1,
                              st).wait()
        pltpu.make_async_copy(dst_hbm.at[pl.ds(wid * ET, ET)], dst1,
                              st).wait()

        @pl.loop(0, NCH)
        def _(t):
            pltpu.make_async_copy(dst_hbm.at[pl.ds(0, 128)], dst2.at[t],
                                  st).wait()

        plsc.subcore_barrier()

        @pl.loop(0, NCH)
        def _(ch):
            for g in range(8):
                base = ch * 128 + g * 16
                sv = src1[pl.ds(base, LN)]
                dv = dst1[pl.ds(base, LN)]
                pv = plsc.load_gather(pt, [sv])
                qv = plsc.load_gather(qt, [dv])
                Lt[pl.ds(base, LN)] = pv + qv
            pltpu.async_copy(Lt.at[pl.ds(ch * 128, 128)],
                             acc_sh.at[dst2.at[ch]], ssm, add=True)

        pltpu.sync_copy(Lt, L_hbm.at[pl.ds(wid * ET, ET)])

        @pl.loop(0, NCH)
        def _(ch):
            pltpu.make_async_copy(Lt.at[pl.ds(0, 128)],
                                  acc_sh.at[dst2.at[0]], ssm).wait()

        plsc.subcore_barrier()
        pltpu.sync_copy(acc_sh.at[pl.ds(sid * stripe, stripe)], zbuf)
        pltpu.sync_copy(zbuf,
                        s_hbm.at[pl.ds(c * NP + sid * stripe, stripe)])

    return k(p, q, srcp, dstp)


def _sc_final(L, srcp, dstp, s2, degc, b3b):
    EP = srcp.shape[0]
    NN = degc.shape[0]
    NP = s2.shape[0] // NC
    ET = EP // NW

    @functools.partial(
        pl.kernel,
        out_type=jax.ShapeDtypeStruct((EP,), F32),
        mesh=_mesh(),
        compiler_params=_sc_params(),
        scratch_types=[
            pltpu.VMEM((NN,), F32),
            pltpu.VMEM((NN,), F32),
            pltpu.VMEM((NN,), F32),
            pltpu.VMEM((ET,), I32),
            pltpu.VMEM((ET,), I32),
            pltpu.VMEM((ET,), F32),
            pltpu.VMEM((ET,), F32),
            pltpu.VMEM((LN,), F32),
            pltpu.SemaphoreType.DMA,
        ],
    )
    def k(L_hbm, src_hbm, dst_hbm, s_hbm, deg_hbm, b3_hbm, out_hbm,
          st, tmp, degt, src1, dst1, Lt, outt, b3t, sem):
        c = lax.axis_index("c")
        sid = lax.axis_index("s")
        wid = c * NS + sid

        copies = [
            (s_hbm.at[pl.ds(0, NN)], st),
            (s_hbm.at[pl.ds(NP, NN)], tmp),
            (deg_hbm, degt),
            (src_hbm.at[pl.ds(wid * ET, ET)], src1),
            (dst_hbm.at[pl.ds(wid * ET, ET)], dst1),
            (L_hbm.at[pl.ds(wid * ET, ET)], Lt),
            (b3_hbm, b3t),
        ]
        for s_ref, d_ref in copies:
            pltpu.async_copy(s_ref, d_ref, sem)
        for s_ref, d_ref in copies:
            pltpu.make_async_copy(s_ref, d_ref, sem).wait()

        b3v = b3t[...]
        one = jnp.ones((LN,), F32)
        zero = jnp.zeros((LN,), F32)

        @pl.loop(0, ET, step=2 * LN)
        def _(j):
            for u in range(2):
                jj = j + u * LN
                sv = src1[pl.ds(jj, LN)]
                dv = dst1[pl.ds(jj, LN)]
                Lv = Lt[pl.ds(jj, LN)]
                selfm = sv == dv
                s_at = (plsc.load_gather(st, [sv])
                        + plsc.load_gather(tmp, [sv]))
                d_at = plsc.load_gather(degt, [sv])
                degv = jnp.maximum(d_at - jnp.where(selfm, one, zero), 1.0)
                num = s_at - jnp.where(selfm, Lv, zero)
                z = Lv + num / degv + b3v
                outt[pl.ds(jj, LN)] = 1.0 / (1.0 + jnp.exp(-z))

        pltpu.sync_copy(outt, out_hbm.at[pl.ds(wid * ET, ET)])

    return k(L, srcp, dstp, s2, degc, b3b)


def kernel(x, edge_index, W1, b1, gamma1, beta1, W2, b2, W3, b3):
    N, D = x.shape
    E = edge_index.shape[1]
    NP = _round_up(N, 256)
    EP = _round_up(E, 128 * NW)

    src = edge_index[0]
    dst = edge_index[1]
    npad = EP - E
    padd = N + (jnp.arange(npad, dtype=I32) % (NP - N))
    pads = jnp.arange(npad, dtype=I32) % (NP - N)
    srcp = jnp.concatenate([src, pads])
    dstp = jnp.concatenate([dst, padd])

    w3a = W3[:D, 0]
    w3b = W3[D:, 0]
    b3b = jnp.broadcast_to(b3, (LN,)).astype(F32)

    asum1, cnt2 = _sc_rowscatter(x, srcp, dstp, NP, with_cnt=True)
    cnt = cnt2.reshape(NC, NP)
    h1, y2, degc = _tc_layer1(x, asum1, cnt, W1, b1, gamma1, beta1, W2, N)
    asum2, = _sc_rowscatter(h1, srcp, dstp, NP)
    p, q = _tc_pq(y2, asum2, degc, b2, W2, w3a, w3b, N)
    L, s2 = _sc_edge_logits(p, q, srcp, dstp, NP)
    outp = _sc_final(L, srcp, dstp, s2, degc, b3b)
    return outp[:E][:, None]
